# Initial kernel scaffold; baseline (speedup 1.0000x reference)
import jax
import jax.numpy as jnp
from jax.experimental import pallas as pl


def kernel(x, edge_index, W1, a_src1, a_dst1, b1, W2, a_src2, a_dst2, b2, P1, pb1, P2, pb2):
    raise NotImplementedError("write your pallas kernel here")


# speedup vs baseline: 1.1269x; 1.1269x over previous
import functools

import jax
import jax.numpy as jnp
from jax.experimental import pallas as pl


def _proj_kernel(t_ref, p1_ref, pb1_ref, p2_ref, pb2_ref, o_ref):
    t = t_ref[...]
    hmid = jnp.maximum(t @ p1_ref[...] + pb1_ref[...], 0.0)
    o_ref[...] = hmid @ p2_ref[...] + pb2_ref[...]


def _proj_pallas(t, P1, pb1, P2, pb2):
    M = t.shape[0]
    lat = P2.shape[1]
    grid = (M // 256,)
    return pl.pallas_call(
        _proj_kernel,
        grid=grid,
        in_specs=[
            pl.BlockSpec((256, t.shape[1]), lambda i: (i, 0)),
            pl.BlockSpec(P1.shape, lambda i: (0, 0)),
            pl.BlockSpec((1, pb1.shape[0]), lambda i: (0, 0)),
            pl.BlockSpec(P2.shape, lambda i: (0, 0)),
            pl.BlockSpec((1, pb2.shape[0]), lambda i: (0, 0)),
        ],
        out_specs=pl.BlockSpec((256, lat), lambda i: (i, 0)),
        out_shape=jax.ShapeDtypeStruct((M, lat), jnp.float32),
    )(t, P1, pb1.reshape(1, -1), P2, pb2.reshape(1, -1))


def _gat_layer(h_in, src, dst, W, a_src, a_dst, b, heads, out_ch):
    N = h_in.shape[0]
    h = (h_in @ W).reshape(N, heads, out_ch)
    as_ = (h * a_src).sum(-1)
    ad_ = (h * a_dst).sum(-1)
    alpha = as_[src] + ad_[dst]
    alpha = jnp.where(alpha >= 0, alpha, 0.2 * alpha)
    e = jnp.exp(alpha)
    denom = jnp.zeros((N, heads), jnp.float32).at[dst].add(e)
    num = jnp.zeros((N, heads, out_ch), jnp.float32).at[dst].add(
        h[src] * e[:, :, None])
    out = num / (denom[:, :, None] + 1e-16)
    return out.reshape(N, heads * out_ch) + b


def kernel(x, edge_index, W1, a_src1, a_dst1, b1, W2, a_src2, a_dst2, b2,
           P1, pb1, P2, pb2):
    N = x.shape[0]
    sl = jnp.arange(N, dtype=edge_index.dtype)
    ei = jnp.concatenate([edge_index, jnp.stack([sl, sl])], axis=1)
    src, dst = ei[0], ei[1]

    h = _gat_layer(x, src, dst, W1, a_src1, a_dst1, b1, 8, 64)
    h = jax.nn.relu(h)
    z = _gat_layer(h, src, dst, W2, a_src2, a_dst2, b2, 1, 128)

    graph_repr = jnp.mean(z, axis=0, keepdims=True)
    rows = jnp.concatenate([z, graph_repr], axis=0)
    Mpad = ((N + 1 + 255) // 256) * 256
    rows = jnp.pad(rows, ((0, Mpad - (N + 1)), (0, 0)))
    out = _proj_pallas(rows, P1, pb1, P2, pb2)
    return (out[:N], out[N:N + 1])


# speedup vs baseline: 18.4738x; 16.3940x over previous
import jax
import jax.numpy as jnp
from jax import lax
from jax.experimental import pallas as pl
from jax.experimental.pallas import tpu as pltpu
from jax.experimental.pallas import tpu_sc as plsc

C = 128
G = C // 16
NT = 16
NC = 2
BLK = 256

_SC_PARAMS = pltpu.CompilerParams(needs_layout_passes=False)


def _zero_1d(ref, n):
    def z(i, _):
        ref[pl.ds(i * 16, 16)] = jnp.zeros((16,), jnp.float32)
        return 0
    lax.fori_loop(0, n // 16, z, 0)


def _leaky_exp(v):
    return jnp.exp(jnp.where(v >= 0.0, v, 0.2 * v))


def _tc0_body(x_ref, w1_ref, ms_ref, md_ref, h0, h1, h2, h3, tsf, tdf):
    h = x_ref[...] @ w1_ref[...]
    for j, r in enumerate((h0, h1, h2, h3)):
        r[...] = h[:, j * 128:(j + 1) * 128]
    tsf[...] = h @ ms_ref[...]
    tdf[...] = h @ md_ref[...]


def _tc0(xp, W1, Ms, Md, npad):
    fb = pl.BlockSpec((BLK, 128), lambda i: (i, 0))
    return pl.pallas_call(
        _tc0_body,
        grid=(npad // BLK,),
        in_specs=[
            pl.BlockSpec((BLK, 128), lambda i: (i, 0)),
            pl.BlockSpec((128, 512), lambda i: (0, 0)),
            pl.BlockSpec((512, 128), lambda i: (0, 0)),
            pl.BlockSpec((512, 128), lambda i: (0, 0)),
        ],
        out_specs=[fb] * 6,
        out_shape=[jax.ShapeDtypeStruct((npad, 128), jnp.float32)] * 6,
    )(xp, W1, Ms, Md)


def _make_sc_alpha1(npad, epad):
    epw = epad // 8
    nch = epw // C
    mesh = plsc.VectorSubcoreMesh(core_axis_name="c", subcore_axis_name="s")

    def body(src_h, dst_h, t0, t1, t2, t3, e0_h, e1_h, e2_h, e3_h,
             d0_h, d1_h, d2_h, d3_h,
             tab_v, denp_v, ebuf, sidx, didx):
        trefs = [t0, t1, t2, t3]
        erefs = [e0_h, e1_h, e2_h, e3_h]
        drefs = [d0_h, d1_h, d2_h, d3_h]
        cid = lax.axis_index("c")
        sid = lax.axis_index("s")
        iota = lax.broadcasted_iota(jnp.int32, (16,), 0)

        def work(tab_hbm, e_hbm, den_hbm, w):
            pltpu.sync_copy(tab_hbm, tab_v)
            _zero_1d(denp_v, 2 * npad)

            def chunk(k, _):
                base = w * epw + k * C
                pltpu.sync_copy(src_h.at[pl.ds(base, C)], sidx)
                pltpu.sync_copy(dst_h.at[pl.ds(base, C)], didx)

                def group(g, _):
                    sv = sidx[pl.ds(g * 16, 16)]
                    dv = didx[pl.ds(g * 16, 16)]
                    a0 = plsc.load_gather(tab_v, [sv * 4])
                    a1 = plsc.load_gather(tab_v, [sv * 4 + 1])
                    b0 = plsc.load_gather(tab_v, [dv * 4 + 2])
                    b1 = plsc.load_gather(tab_v, [dv * 4 + 3])
                    e0v = _leaky_exp(a0 + b0)
                    e1v = _leaky_exp(a1 + b1)
                    ebuf[pl.ds(g * 16, 16)] = e0v
                    ebuf[pl.ds(C + g * 16, 16)] = e1v
                    for l in range(16):
                        e0b = jnp.full((16,), e0v[l], jnp.float32)
                        e1b = jnp.full((16,), e1v[l], jnp.float32)
                        di = (jnp.full((16,), dv[l], jnp.int32)
                              + (iota & 1) * npad)
                        vals = jnp.where(iota == 0, e0b, e1b)
                        plsc.addupdate_scatter(denp_v, [di], vals,
                                               mask=iota < 2)
                    return 0
                lax.fori_loop(0, G, group, 0)
                pltpu.sync_copy(ebuf, e_hbm.at[pl.ds(2 * base, 2 * C)])
                return 0
            lax.fori_loop(0, nch, chunk, 0)
            pltpu.sync_copy(denp_v,
                            den_hbm.at[pl.ds(w * 2 * npad, 2 * npad)])

        for cc in range(NC):
            @pl.when(cid == cc)
            def _():
                for half in range(2):
                    hp = cc * 2 + half
                    pred = (sid < 8) if half == 0 else (sid >= 8)

                    @pl.when(pred)
                    def _():
                        work(trefs[hp], erefs[hp], drefs[hp],
                             sid - half * 8)

    return pl.kernel(
        body,
        out_type=[jax.ShapeDtypeStruct((2 * epad,), jnp.float32)] * 4
        + [jax.ShapeDtypeStruct((8 * 2 * npad,), jnp.float32)] * 4,
        mesh=mesh,
        compiler_params=_SC_PARAMS,
        scratch_types=[
            pltpu.VMEM((4 * npad,), jnp.float32),
            pltpu.VMEM((2 * npad,), jnp.float32),
            pltpu.VMEM((2 * C,), jnp.float32),
            pltpu.VMEM((C,), jnp.int32),
            pltpu.VMEM((C,), jnp.int32),
        ],
    )


def _tcr1_body(d0, d1, d2, d3, r0, r1, r2, r3):
    for dref, rref in zip((d0, d1, d2, d3), (r0, r1, r2, r3)):
        s = jnp.sum(dref[...], axis=0)
        rref[...] = 1.0 / (s + 1e-16)


def _tcr1(dens, npad):
    db = pl.BlockSpec((8, 2, BLK), lambda i: (0, 0, i))
    rb = pl.BlockSpec((2, BLK), lambda i: (0, i))
    return pl.pallas_call(
        _tcr1_body,
        grid=(npad // BLK,),
        in_specs=[db] * 4,
        out_specs=[rb] * 4,
        out_shape=[jax.ShapeDtypeStruct((2, npad), jnp.float32)] * 4,
    )(*[d.reshape(8, 2, npad) for d in dens])


def _make_sc_agg1(npad, epad):
    rpt = npad // NT
    nzc = rpt // 64
    nch = epad // C // NT
    mesh = plsc.VectorSubcoreMesh(core_axis_name="c", subcore_axis_name="s")

    def body(src_h, dst_h, h0, h1, h2, h3, e0_h, e1_h, e2_h, e3_h,
             r0_h, r1_h, r2_h, r3_h, num0, num1, num2, num3,
             acc_sh, rec0_v, rec1_v, ebuf, sidx, didx, rows, buf, sem_r):
        hrefs = [h0, h1, h2, h3]
        erefs = [e0_h, e1_h, e2_h, e3_h]
        rrefs = [r0_h, r1_h, r2_h, r3_h]
        numrefs = [num0, num1, num2, num3]
        cid = lax.axis_index("c")
        sid = lax.axis_index("s")
        zbase = sid * rpt

        def job(rows_hbm, e_hbm, rec_hbm, num_hbm):
            pltpu.sync_copy(rec_hbm.at[pl.ds(sid * rpt, rpt)], rec0_v)
            pltpu.sync_copy(rec_hbm.at[pl.ds(npad + sid * rpt, rpt)],
                            rec1_v)
            def zcp(i, _):
                pltpu.sync_copy(buf, acc_sh.at[pl.ds(zbase + i * 64, 64)])
                return 0

            def zrow(i, _):
                for v in range(8):
                    buf[i, pl.ds(v * 16, 16)] = jnp.zeros((16,),
                                                          jnp.float32)
                return 0
            lax.fori_loop(0, 64, zrow, 0)
            lax.fori_loop(0, nzc, zcp, 0)
            plsc.subcore_barrier()

            def chunk(k, _):
                base = (sid * nch + k) * C
                pltpu.sync_copy(src_h.at[pl.ds(base, C)], sidx)
                pltpu.sync_copy(dst_h.at[pl.ds(base, C)], didx)
                pltpu.sync_copy(e_hbm.at[pl.ds(2 * base, 2 * C)], ebuf)
                pltpu.async_copy(rows_hbm.at[sidx], rows, sem_r).wait()

                def group(g, _):
                    e0v = ebuf[pl.ds(g * 16, 16)]
                    e1v = ebuf[pl.ds(C + g * 16, 16)]
                    for l in range(16):
                        e0 = jnp.full((16,), e0v[l], jnp.float32)
                        e1 = jnp.full((16,), e1v[l], jnp.float32)
                        r = g * 16 + l
                        for v in range(8):
                            m = e0 if v < 4 else e1
                            rows[r, pl.ds(v * 16, 16)] = (
                                rows[r, pl.ds(v * 16, 16)] * m)
                    return 0
                lax.fori_loop(0, G, group, 0)
                pltpu.sync_copy(rows, acc_sh.at[didx], add=True)
                return 0
            lax.fori_loop(0, nch, chunk, 0)
            plsc.subcore_barrier()

            def wcp(i, _):
                pltpu.sync_copy(acc_sh.at[pl.ds(zbase + i * 64, 64)], buf)

                def q8(q, _):
                    rc0 = rec0_v[pl.ds(i * 64 + q * 16, 16)]
                    rc1 = rec1_v[pl.ds(i * 64 + q * 16, 16)]
                    for l in range(16):
                        m0 = jnp.full((16,), rc0[l], jnp.float32)
                        m1 = jnp.full((16,), rc1[l], jnp.float32)
                        row = q * 16 + l
                        for v in range(8):
                            m = m0 if v < 4 else m1
                            buf[row, pl.ds(v * 16, 16)] = (
                                buf[row, pl.ds(v * 16, 16)] * m)
                    return 0
                lax.fori_loop(0, 4, q8, 0)
                pltpu.sync_copy(buf, num_hbm.at[pl.ds(zbase + i * 64, 64)])
                return 0
            lax.fori_loop(0, nzc, wcp, 0)
            plsc.subcore_barrier()

        for cc in range(NC):
            @pl.when(cid == cc)
            def _():
                for jj in range(2):
                    hp = cc * 2 + jj
                    job(hrefs[hp], erefs[hp], rrefs[hp], numrefs[hp])

    return pl.kernel(
        body,
        out_type=[jax.ShapeDtypeStruct((npad, 128), jnp.float32)] * 4,
        mesh=mesh,
        compiler_params=_SC_PARAMS,
        scratch_types=[
            pltpu.VMEM_SHARED((npad, 128), jnp.float32),
            pltpu.VMEM((npad // NT,), jnp.float32),
            pltpu.VMEM((npad // NT,), jnp.float32),
            pltpu.VMEM((2 * C,), jnp.float32),
            pltpu.VMEM((C,), jnp.int32),
            pltpu.VMEM((C,), jnp.int32),
            pltpu.VMEM((C, 128), jnp.float32),
            pltpu.VMEM((64, 128), jnp.float32),
            pltpu.SemaphoreType.DMA,
        ],
    )


def _make_sc_alpha2(npad, epad):
    epw = epad // (NT * NC)
    nch = epw // C
    mesh = plsc.VectorSubcoreMesh(core_axis_name="c", subcore_axis_name="s")

    def body(src_h, dst_h, ts_h, td_h, e_h, den_h,
             ts_v, td_v, denp_v, ebuf, sidx, didx):
        cid = lax.axis_index("c")
        sid = lax.axis_index("s")
        wid = cid * NT + sid
        iota = lax.broadcasted_iota(jnp.int32, (16,), 0)
        pltpu.sync_copy(ts_h, ts_v)
        pltpu.sync_copy(td_h, td_v)
        _zero_1d(denp_v, npad)

        def chunk(k, _):
            base = (wid * nch + k) * C
            pltpu.sync_copy(src_h.at[pl.ds(base, C)], sidx)
            pltpu.sync_copy(dst_h.at[pl.ds(base, C)], didx)

            def group(g, _):
                sv = sidx[pl.ds(g * 16, 16)]
                dv = didx[pl.ds(g * 16, 16)]
                a = plsc.load_gather(ts_v, [sv])
                b = plsc.load_gather(td_v, [dv])
                ev = _leaky_exp(a + b)
                ebuf[pl.ds(g * 16, 16)] = ev
                for l in range(16):
                    eb = jnp.full((16,), ev[l], jnp.float32)
                    di = jnp.full((16,), dv[l], jnp.int32)
                    plsc.addupdate_scatter(denp_v, [di], eb,
                                           mask=iota < 1)
                return 0
            lax.fori_loop(0, G, group, 0)
            pltpu.sync_copy(ebuf, e_h.at[pl.ds(base, C)])
            return 0
        lax.fori_loop(0, nch, chunk, 0)
        pltpu.sync_copy(denp_v, den_h.at[pl.ds(wid * npad, npad)])

    return pl.kernel(
        body,
        out_type=[jax.ShapeDtypeStruct((epad,), jnp.float32),
                  jax.ShapeDtypeStruct((NT * NC * npad,), jnp.float32)],
        mesh=mesh,
        compiler_params=_SC_PARAMS,
        scratch_types=[
            pltpu.VMEM((npad,), jnp.float32),
            pltpu.VMEM((npad,), jnp.float32),
            pltpu.VMEM((npad,), jnp.float32),
            pltpu.VMEM((C,), jnp.float32),
            pltpu.VMEM((C,), jnp.int32),
            pltpu.VMEM((C,), jnp.int32),
        ],
    )


def _tcr2_body(d_ref, r_ref):
    s = jnp.sum(d_ref[...], axis=0)
    r_ref[...] = (1.0 / (s + 1e-16)).reshape(1, BLK)


def _tcr2(den, npad):
    return pl.pallas_call(
        _tcr2_body,
        grid=(npad // BLK,),
        in_specs=[pl.BlockSpec((NT * NC, BLK), lambda i: (0, i))],
        out_specs=pl.BlockSpec((1, BLK), lambda i: (0, i)),
        out_shape=jax.ShapeDtypeStruct((1, npad), jnp.float32),
    )(den.reshape(NT * NC, npad))


def _make_sc_agg2(npad, epad):
    rpt = npad // NT
    nzc = rpt // 64
    nch = epad // C // (NT * NC)
    mesh = plsc.VectorSubcoreMesh(core_axis_name="c", subcore_axis_name="s")

    def body(src_h, dst_h, h2t, e_h, rec_h, numa, numb,
             acc_sh, rec_v, ebuf, sidx, didx, rows, buf, sem_r):
        cid = lax.axis_index("c")
        sid = lax.axis_index("s")
        zbase = sid * rpt
        wid = cid * NT + sid
        pltpu.sync_copy(rec_h.at[pl.ds(sid * rpt, rpt)], rec_v)

        def zrow(i, _):
            for v in range(8):
                buf[i, pl.ds(v * 16, 16)] = jnp.zeros((16,), jnp.float32)
            return 0
        lax.fori_loop(0, 64, zrow, 0)

        def zcp(i, _):
            pltpu.sync_copy(buf, acc_sh.at[pl.ds(zbase + i * 64, 64)])
            return 0
        lax.fori_loop(0, nzc, zcp, 0)
        plsc.subcore_barrier()

        def chunk(k, _):
            base = (wid * nch + k) * C
            pltpu.sync_copy(src_h.at[pl.ds(base, C)], sidx)
            pltpu.sync_copy(dst_h.at[pl.ds(base, C)], didx)
            pltpu.sync_copy(e_h.at[pl.ds(base, C)], ebuf)
            pltpu.async_copy(h2t.at[sidx], rows, sem_r).wait()

            def group(g, _):
                ev = ebuf[pl.ds(g * 16, 16)]
                for l in range(16):
                    e = jnp.full((16,), ev[l], jnp.float32)
                    r = g * 16 + l
                    for v in range(8):
                        rows[r, pl.ds(v * 16, 16)] = (
                            rows[r, pl.ds(v * 16, 16)] * e)
                return 0
            lax.fori_loop(0, G, group, 0)
            pltpu.sync_copy(rows, acc_sh.at[didx], add=True)
            return 0
        lax.fori_loop(0, nch, chunk, 0)
        plsc.subcore_barrier()

        def wcp_core(num_hbm):
            def wcp(i, _):
                pltpu.sync_copy(acc_sh.at[pl.ds(zbase + i * 64, 64)], buf)

                def q16(q, _):
                    rc = rec_v[pl.ds(i * 64 + q * 16, 16)]
                    for l in range(16):
                        m = jnp.full((16,), rc[l], jnp.float32)
                        row = q * 16 + l
                        for v in range(8):
                            buf[row, pl.ds(v * 16, 16)] = (
                                buf[row, pl.ds(v * 16, 16)] * m)
                    return 0
                lax.fori_loop(0, 4, q16, 0)
                pltpu.sync_copy(buf, num_hbm.at[pl.ds(zbase + i * 64, 64)])
                return 0
            lax.fori_loop(0, nzc, wcp, 0)

        for cc in range(NC):
            @pl.when(cid == cc)
            def _():
                wcp_core([numa, numb][cc])

    return pl.kernel(
        body,
        out_type=[jax.ShapeDtypeStruct((npad, 128), jnp.float32)] * 2,
        mesh=mesh,
        compiler_params=_SC_PARAMS,
        scratch_types=[
            pltpu.VMEM_SHARED((npad, 128), jnp.float32),
            pltpu.VMEM((npad // NT,), jnp.float32),
            pltpu.VMEM((C,), jnp.float32),
            pltpu.VMEM((C,), jnp.int32),
            pltpu.VMEM((C,), jnp.int32),
            pltpu.VMEM((C, 128), jnp.float32),
            pltpu.VMEM((64, 128), jnp.float32),
            pltpu.SemaphoreType.DMA,
        ],
    )


def _tc1_body(n0, n1, n2, n3, b1_ref, w2_ref, ms2_ref, md2_ref,
              h2_ref, tsf_ref, tdf_ref):
    h_mid = jnp.concatenate(
        [n0[...], n1[...], n2[...], n3[...]], axis=1) + b1_ref[...]
    h_mid = jnp.maximum(h_mid, 0.0)
    h2 = h_mid @ w2_ref[...]
    h2_ref[...] = h2
    tsf_ref[...] = h2 @ ms2_ref[...]
    tdf_ref[...] = h2 @ md2_ref[...]


def _tc1(nums, b1, W2, Ms2, Md2, npad):
    fb = pl.BlockSpec((BLK, 128), lambda i: (i, 0))
    return pl.pallas_call(
        _tc1_body,
        grid=(npad // BLK,),
        in_specs=[fb] * 4 + [
            pl.BlockSpec((1, 512), lambda i: (0, 0)),
            pl.BlockSpec((512, 128), lambda i: (0, 0)),
            pl.BlockSpec((128, 128), lambda i: (0, 0)),
            pl.BlockSpec((128, 128), lambda i: (0, 0)),
        ],
        out_specs=[fb, fb, fb],
        out_shape=[jax.ShapeDtypeStruct((npad, 128), jnp.float32)] * 3,
    )(*nums, b1.reshape(1, 512), W2, Ms2, Md2)


def _make_tc2_body(n_real, nblocks):
    def body(na, nb, b2_ref, p1_ref, pb1_ref, p2_ref, pb2_ref,
             zp_ref, gsum_ref, gp_ref):
        i = pl.program_id(0)
        z = na[...] + nb[...] + b2_ref[...]
        zp_ref[...] = jnp.maximum(z @ p1_ref[...] + pb1_ref[...],
                                  0.0) @ p2_ref[...] + pb2_ref[...]
        rid = BLK * i + lax.broadcasted_iota(jnp.int32, (BLK, 1), 0)
        part = jnp.sum(jnp.where(rid < n_real, z, 0.0), axis=0,
                       keepdims=True)

        @pl.when(i == 0)
        def _():
            gsum_ref[...] = part

        @pl.when(i > 0)
        def _():
            gsum_ref[...] = gsum_ref[...] + part

        @pl.when(i == nblocks - 1)
        def _():
            g = gsum_ref[...] / float(n_real)
            gp_ref[...] = jnp.maximum(g @ p1_ref[...] + pb1_ref[...],
                                      0.0) @ p2_ref[...] + pb2_ref[...]
    return body


def _tc2(numa, numb, b2, P1, pb1, P2, pb2, n_real, npad):
    nblocks = npad // BLK
    fb = pl.BlockSpec((BLK, 128), lambda i: (i, 0))
    one = pl.BlockSpec((1, 128), lambda i: (0, 0))
    return pl.pallas_call(
        _make_tc2_body(n_real, nblocks),
        grid=(nblocks,),
        in_specs=[fb, fb,
                  pl.BlockSpec((1, 128), lambda i: (0, 0)),
                  pl.BlockSpec((128, 64), lambda i: (0, 0)),
                  pl.BlockSpec((1, 64), lambda i: (0, 0)),
                  pl.BlockSpec((64, 128), lambda i: (0, 0)),
                  pl.BlockSpec((1, 128), lambda i: (0, 0))],
        out_specs=[fb, one, one],
        out_shape=[jax.ShapeDtypeStruct((npad, 128), jnp.float32),
                   jax.ShapeDtypeStruct((1, 128), jnp.float32),
                   jax.ShapeDtypeStruct((1, 128), jnp.float32)],
    )(numa, numb, b2.reshape(1, 128), P1, pb1.reshape(1, 64),
      P2, pb2.reshape(1, 128))


def _logit_mat(a, heads, ch):
    af = a.reshape(heads, ch)
    cols = []
    for c in range(128):
        h_ = (c // 16) % heads
        v = jnp.zeros((heads * ch,), jnp.float32)
        v = v.at[ch * h_:ch * (h_ + 1)].set(af[h_])
        cols.append(v)
    return jnp.stack(cols, axis=1)


def kernel(x, edge_index, W1, a_src1, a_dst1, b1, W2, a_src2, a_dst2, b2,
           P1, pb1, P2, pb2):
    N, _ = x.shape
    E = edge_index.shape[1]
    npad = ((N + (64 * NT) - 1) // (64 * NT)) * (64 * NT)
    gran = C * NT * NC
    epad = ((E + N + gran - 1) // gran) * gran

    xp = jnp.pad(x, ((0, npad - N), (0, 0)))
    sl = jnp.arange(N, dtype=jnp.int32)
    pad_n = epad - E - N
    src = jnp.concatenate(
        [edge_index[0].astype(jnp.int32), sl,
         jnp.zeros((pad_n,), jnp.int32)])
    dst = jnp.concatenate(
        [edge_index[1].astype(jnp.int32), sl,
         jnp.full((pad_n,), N, jnp.int32)])

    Ms = _logit_mat(a_src1, 8, 64)
    Md = _logit_mat(a_dst1, 8, 64)
    h1p0, h1p1, h1p2, h1p3, tsf, tdf = _tc0(xp, W1, Ms, Md, npad)

    tabs = []
    for j in range(4):
        t = jnp.stack([tsf[:, 32 * j], tsf[:, 32 * j + 16],
                       tdf[:, 32 * j], tdf[:, 32 * j + 16]], axis=1)
        tabs.append(t.reshape(-1))

    a1 = _make_sc_alpha1(npad, epad)(src, dst, *tabs)
    evals1, dens1 = a1[0:4], a1[4:8]
    recs1 = _tcr1(dens1, npad)
    recs1 = [r.reshape(-1) for r in recs1]

    sc_agg1 = _make_sc_agg1(npad, epad)
    nums = sc_agg1(src, dst, h1p0, h1p1, h1p2, h1p3, *evals1, *recs1)

    Ms2 = jnp.tile(a_src2.reshape(128, 1), (1, 128))
    Md2 = jnp.tile(a_dst2.reshape(128, 1), (1, 128))
    h2, tsf2, tdf2 = _tc1(nums, b1, W2, Ms2, Md2, npad)

    a2 = _make_sc_alpha2(npad, epad)(src, dst, tsf2[:, 0], tdf2[:, 0])
    evals2, dens2 = a2
    rec2 = _tcr2(dens2, npad).reshape(-1)

    numa, numb = _make_sc_agg2(npad, epad)(src, dst, h2, evals2, rec2)

    zp, _, gp = _tc2(numa, numb, b2, P1, pb1, P2, pb2, N, npad)
    return (zp[:N], gp)


# speedup vs baseline: 22.7573x; 1.2319x over previous
import jax
import jax.numpy as jnp
from jax import lax
from jax.experimental import pallas as pl
from jax.experimental.pallas import tpu as pltpu
from jax.experimental.pallas import tpu_sc as plsc

C = 128
G = C // 16
NT = 16
NC = 2
BLK = 256

_SC_PARAMS = pltpu.CompilerParams(needs_layout_passes=False)


def _zero_1d(ref, n):
    def z(i, _):
        ref[pl.ds(i * 16, 16)] = jnp.zeros((16,), jnp.float32)
        return 0
    lax.fori_loop(0, n // 16, z, 0)


def _leaky_exp(v):
    return jnp.exp(jnp.where(v >= 0.0, v, 0.2 * v))


def _tc0_body(x_ref, w1_ref, ms_ref, md_ref, h0, h1, h2, h3, tsf, tdf):
    h = x_ref[...] @ w1_ref[...]
    for j, r in enumerate((h0, h1, h2, h3)):
        r[...] = h[:, j * 128:(j + 1) * 128]
    tsf[...] = h @ ms_ref[...]
    tdf[...] = h @ md_ref[...]


def _tc0(xp, W1, Ms, Md, npad):
    fb = pl.BlockSpec((BLK, 128), lambda i: (i, 0))
    return pl.pallas_call(
        _tc0_body,
        grid=(npad // BLK,),
        in_specs=[
            pl.BlockSpec((BLK, 128), lambda i: (i, 0)),
            pl.BlockSpec((128, 512), lambda i: (0, 0)),
            pl.BlockSpec((512, 128), lambda i: (0, 0)),
            pl.BlockSpec((512, 128), lambda i: (0, 0)),
        ],
        out_specs=[fb] * 6,
        out_shape=[jax.ShapeDtypeStruct((npad, 128), jnp.float32)] * 6,
    )(xp, W1, Ms, Md)


def _make_sc_alpha1(npad, epad):
    epw = epad // 8
    nch = epw // C
    mesh = plsc.VectorSubcoreMesh(core_axis_name="c", subcore_axis_name="s")

    def body(src_h, dst_h, t0, t1, t2, t3, e0_h, e1_h, e2_h, e3_h,
             d0_h, d1_h, d2_h, d3_h,
             tab_v, denp_v, ebuf, sidx, didx):
        trefs = [t0, t1, t2, t3]
        erefs = [e0_h, e1_h, e2_h, e3_h]
        drefs = [d0_h, d1_h, d2_h, d3_h]
        cid = lax.axis_index("c")
        sid = lax.axis_index("s")
        iota = lax.broadcasted_iota(jnp.int32, (16,), 0)

        def work(tab_hbm, e_hbm, den_hbm, w):
            pltpu.sync_copy(tab_hbm, tab_v)
            _zero_1d(denp_v, 2 * npad)

            def chunk(k, _):
                base = w * epw + k * C
                pltpu.sync_copy(src_h.at[pl.ds(base, C)], sidx)
                pltpu.sync_copy(dst_h.at[pl.ds(base, C)], didx)

                def group(g, _):
                    sv = sidx[pl.ds(g * 16, 16)]
                    dv = didx[pl.ds(g * 16, 16)]
                    a0 = plsc.load_gather(tab_v, [sv * 4])
                    a1 = plsc.load_gather(tab_v, [sv * 4 + 1])
                    b0 = plsc.load_gather(tab_v, [dv * 4 + 2])
                    b1 = plsc.load_gather(tab_v, [dv * 4 + 3])
                    e0v = _leaky_exp(a0 + b0)
                    e1v = _leaky_exp(a1 + b1)
                    ebuf[pl.ds(g * 16, 16)] = e0v
                    ebuf[pl.ds(C + g * 16, 16)] = e1v
                    for l in range(16):
                        e0b = jnp.full((16,), e0v[l], jnp.float32)
                        e1b = jnp.full((16,), e1v[l], jnp.float32)
                        di = (jnp.full((16,), dv[l], jnp.int32)
                              + (iota & 1) * npad)
                        vals = jnp.where(iota == 0, e0b, e1b)
                        plsc.addupdate_scatter(denp_v, [di], vals,
                                               mask=iota < 2)
                    return 0
                lax.fori_loop(0, G, group, 0)
                pltpu.sync_copy(ebuf, e_hbm.at[pl.ds(2 * base, 2 * C)])
                return 0
            lax.fori_loop(0, nch, chunk, 0)
            pltpu.sync_copy(denp_v,
                            den_hbm.at[pl.ds(w * 2 * npad, 2 * npad)])

        for cc in range(NC):
            @pl.when(cid == cc)
            def _():
                for half in range(2):
                    hp = cc * 2 + half
                    pred = (sid < 8) if half == 0 else (sid >= 8)

                    @pl.when(pred)
                    def _():
                        work(trefs[hp], erefs[hp], drefs[hp],
                             sid - half * 8)

    return pl.kernel(
        body,
        out_type=[jax.ShapeDtypeStruct((2 * epad,), jnp.float32)] * 4
        + [jax.ShapeDtypeStruct((8 * 2 * npad,), jnp.float32)] * 4,
        mesh=mesh,
        compiler_params=_SC_PARAMS,
        scratch_types=[
            pltpu.VMEM((4 * npad,), jnp.float32),
            pltpu.VMEM((2 * npad,), jnp.float32),
            pltpu.VMEM((2 * C,), jnp.float32),
            pltpu.VMEM((C,), jnp.int32),
            pltpu.VMEM((C,), jnp.int32),
        ],
    )


def _tcr1_body(d0, d1, d2, d3, r0, r1, r2, r3):
    for dref, rref in zip((d0, d1, d2, d3), (r0, r1, r2, r3)):
        s = jnp.sum(dref[...], axis=0)
        rref[...] = 1.0 / (s + 1e-16)


def _tcr1(dens, npad):
    db = pl.BlockSpec((8, 2, BLK), lambda i: (0, 0, i))
    rb = pl.BlockSpec((2, BLK), lambda i: (0, i))
    return pl.pallas_call(
        _tcr1_body,
        grid=(npad // BLK,),
        in_specs=[db] * 4,
        out_specs=[rb] * 4,
        out_shape=[jax.ShapeDtypeStruct((2, npad), jnp.float32)] * 4,
    )(*[d.reshape(8, 2, npad) for d in dens])


def _make_sc_agg1(npad, epad):
    rpt = npad // NT
    nzc = rpt // 64
    nch = epad // C // NT
    mesh = plsc.VectorSubcoreMesh(core_axis_name="c", subcore_axis_name="s")

    def body(src_h, dst_h, h0, h1, h2, h3, e0_h, e1_h, e2_h, e3_h,
             r0_h, r1_h, r2_h, r3_h, num0, num1, num2, num3,
             acc_sh, rec0_v, rec1_v, ebuf_a, ebuf_b, sidx_a, sidx_b,
             didx_a, didx_b, rows_a, rows_b, buf, sem_a, sem_b):
        hrefs = [h0, h1, h2, h3]
        erefs = [e0_h, e1_h, e2_h, e3_h]
        rrefs = [r0_h, r1_h, r2_h, r3_h]
        numrefs = [num0, num1, num2, num3]
        cid = lax.axis_index("c")
        sid = lax.axis_index("s")
        zbase = sid * rpt
        sems = [sem_a, sem_b]
        ebufs = [ebuf_a, ebuf_b]
        sidxs = [sidx_a, sidx_b]
        didxs = [didx_a, didx_b]
        rowss = [rows_a, rows_b]

        def job(rows_hbm, e_hbm, rec_hbm, num_hbm):
            pltpu.sync_copy(rec_hbm.at[pl.ds(sid * rpt, rpt)], rec0_v)
            pltpu.sync_copy(rec_hbm.at[pl.ds(npad + sid * rpt, rpt)],
                            rec1_v)
            def zcp(i, _):
                pltpu.sync_copy(buf, acc_sh.at[pl.ds(zbase + i * 64, 64)])
                return 0

            def zrow(i, _):
                for v in range(8):
                    buf[i, pl.ds(v * 16, 16)] = jnp.zeros((16,),
                                                          jnp.float32)
                return 0
            lax.fori_loop(0, 64, zrow, 0)
            lax.fori_loop(0, nzc, zcp, 0)
            plsc.subcore_barrier()

            def start(k, b):
                base = (sid * nch + k) * C
                pltpu.sync_copy(src_h.at[pl.ds(base, C)], sidxs[b])
                pltpu.sync_copy(dst_h.at[pl.ds(base, C)], didxs[b])
                pltpu.sync_copy(e_hbm.at[pl.ds(2 * base, 2 * C)],
                                ebufs[b])
                pltpu.make_async_copy(rows_hbm.at[sidxs[b]],
                                      rowss[b], sems[b]).start()

            def finish(b):
                pltpu.make_async_copy(rows_hbm.at[sidxs[b]],
                                      rowss[b], sems[b]).wait()
                rows = rowss[b]
                ebuf = ebufs[b]

                def group(g, _):
                    e0v = ebuf[pl.ds(g * 16, 16)]
                    e1v = ebuf[pl.ds(C + g * 16, 16)]
                    for l in range(16):
                        e0 = jnp.full((16,), e0v[l], jnp.float32)
                        e1 = jnp.full((16,), e1v[l], jnp.float32)
                        r = g * 16 + l
                        for v in range(8):
                            m = e0 if v < 4 else e1
                            rows[r, pl.ds(v * 16, 16)] = (
                                rows[r, pl.ds(v * 16, 16)] * m)
                    return 0
                lax.fori_loop(0, G, group, 0)
                pltpu.sync_copy(rows, acc_sh.at[didxs[b]], add=True)

            start(0, 0)

            def pair(p, _):
                start(2 * p + 1, 1)
                finish(0)

                @pl.when(p < nch // 2 - 1)
                def _():
                    start(2 * p + 2, 0)
                finish(1)
                return 0
            lax.fori_loop(0, nch // 2, pair, 0)
            plsc.subcore_barrier()

            def wcp(i, _):
                pltpu.sync_copy(acc_sh.at[pl.ds(zbase + i * 64, 64)], buf)

                def q8(q, _):
                    rc0 = rec0_v[pl.ds(i * 64 + q * 16, 16)]
                    rc1 = rec1_v[pl.ds(i * 64 + q * 16, 16)]
                    for l in range(16):
                        m0 = jnp.full((16,), rc0[l], jnp.float32)
                        m1 = jnp.full((16,), rc1[l], jnp.float32)
                        row = q * 16 + l
                        for v in range(8):
                            m = m0 if v < 4 else m1
                            buf[row, pl.ds(v * 16, 16)] = (
                                buf[row, pl.ds(v * 16, 16)] * m)
                    return 0
                lax.fori_loop(0, 4, q8, 0)
                pltpu.sync_copy(buf, num_hbm.at[pl.ds(zbase + i * 64, 64)])
                return 0
            lax.fori_loop(0, nzc, wcp, 0)
            plsc.subcore_barrier()

        for cc in range(NC):
            @pl.when(cid == cc)
            def _():
                for jj in range(2):
                    hp = cc * 2 + jj
                    job(hrefs[hp], erefs[hp], rrefs[hp], numrefs[hp])

    return pl.kernel(
        body,
        out_type=[jax.ShapeDtypeStruct((npad, 128), jnp.float32)] * 4,
        mesh=mesh,
        compiler_params=_SC_PARAMS,
        scratch_types=[
            pltpu.VMEM_SHARED((npad, 128), jnp.float32),
            pltpu.VMEM((npad // NT,), jnp.float32),
            pltpu.VMEM((npad // NT,), jnp.float32),
            pltpu.VMEM((2 * C,), jnp.float32),
            pltpu.VMEM((2 * C,), jnp.float32),
            pltpu.VMEM((C,), jnp.int32),
            pltpu.VMEM((C,), jnp.int32),
            pltpu.VMEM((C,), jnp.int32),
            pltpu.VMEM((C,), jnp.int32),
            pltpu.VMEM((C, 128), jnp.float32),
            pltpu.VMEM((C, 128), jnp.float32),
            pltpu.VMEM((64, 128), jnp.float32),
            pltpu.SemaphoreType.DMA,
            pltpu.SemaphoreType.DMA,
        ],
    )


def _make_sc_alpha2(npad, epad):
    epw = epad // (NT * NC)
    nch = epw // C
    mesh = plsc.VectorSubcoreMesh(core_axis_name="c", subcore_axis_name="s")

    def body(src_h, dst_h, ts_h, td_h, e_h, den_h,
             ts_v, td_v, denp_v, ebuf, sidx, didx):
        cid = lax.axis_index("c")
        sid = lax.axis_index("s")
        wid = cid * NT + sid
        iota = lax.broadcasted_iota(jnp.int32, (16,), 0)
        pltpu.sync_copy(ts_h, ts_v)
        pltpu.sync_copy(td_h, td_v)
        _zero_1d(denp_v, npad)

        def chunk(k, _):
            base = (wid * nch + k) * C
            pltpu.sync_copy(src_h.at[pl.ds(base, C)], sidx)
            pltpu.sync_copy(dst_h.at[pl.ds(base, C)], didx)

            def group(g, _):
                sv = sidx[pl.ds(g * 16, 16)]
                dv = didx[pl.ds(g * 16, 16)]
                a = plsc.load_gather(ts_v, [sv])
                b = plsc.load_gather(td_v, [dv])
                ev = _leaky_exp(a + b)
                ebuf[pl.ds(g * 16, 16)] = ev
                for l in range(16):
                    eb = jnp.full((16,), ev[l], jnp.float32)
                    di = jnp.full((16,), dv[l], jnp.int32)
                    plsc.addupdate_scatter(denp_v, [di], eb,
                                           mask=iota < 1)
                return 0
            lax.fori_loop(0, G, group, 0)
            pltpu.sync_copy(ebuf, e_h.at[pl.ds(base, C)])
            return 0
        lax.fori_loop(0, nch, chunk, 0)
        pltpu.sync_copy(denp_v, den_h.at[pl.ds(wid * npad, npad)])

    return pl.kernel(
        body,
        out_type=[jax.ShapeDtypeStruct((epad,), jnp.float32),
                  jax.ShapeDtypeStruct((NT * NC * npad,), jnp.float32)],
        mesh=mesh,
        compiler_params=_SC_PARAMS,
        scratch_types=[
            pltpu.VMEM((npad,), jnp.float32),
            pltpu.VMEM((npad,), jnp.float32),
            pltpu.VMEM((npad,), jnp.float32),
            pltpu.VMEM((C,), jnp.float32),
            pltpu.VMEM((C,), jnp.int32),
            pltpu.VMEM((C,), jnp.int32),
        ],
    )


def _tcr2_body(d_ref, r_ref):
    s = jnp.sum(d_ref[...], axis=0)
    r_ref[...] = (1.0 / (s + 1e-16)).reshape(1, BLK)


def _tcr2(den, npad):
    return pl.pallas_call(
        _tcr2_body,
        grid=(npad // BLK,),
        in_specs=[pl.BlockSpec((NT * NC, BLK), lambda i: (0, i))],
        out_specs=pl.BlockSpec((1, BLK), lambda i: (0, i)),
        out_shape=jax.ShapeDtypeStruct((1, npad), jnp.float32),
    )(den.reshape(NT * NC, npad))


def _make_sc_agg2(npad, epad):
    rpt = npad // NT
    nzc = rpt // 64
    nch = epad // C // (NT * NC)
    mesh = plsc.VectorSubcoreMesh(core_axis_name="c", subcore_axis_name="s")

    def body(src_h, dst_h, h2t, e_h, rec_h, numa, numb,
             acc_sh, rec_v, ebuf_a, ebuf_b, sidx_a, sidx_b,
             didx_a, didx_b, rows_a, rows_b, buf,
             sem_a, sem_b):
        cid = lax.axis_index("c")
        sid = lax.axis_index("s")
        zbase = sid * rpt
        wid = cid * NT + sid
        sems = [sem_a, sem_b]
        ebufs = [ebuf_a, ebuf_b]
        sidxs = [sidx_a, sidx_b]
        didxs = [didx_a, didx_b]
        rowss = [rows_a, rows_b]
        pltpu.sync_copy(rec_h.at[pl.ds(sid * rpt, rpt)], rec_v)

        def zrow(i, _):
            for v in range(8):
                buf[i, pl.ds(v * 16, 16)] = jnp.zeros((16,), jnp.float32)
            return 0
        lax.fori_loop(0, 64, zrow, 0)

        def zcp(i, _):
            pltpu.sync_copy(buf, acc_sh.at[pl.ds(zbase + i * 64, 64)])
            return 0
        lax.fori_loop(0, nzc, zcp, 0)
        plsc.subcore_barrier()

        def start(k, b):
            base = (wid * nch + k) * C
            pltpu.sync_copy(src_h.at[pl.ds(base, C)], sidxs[b])
            pltpu.sync_copy(dst_h.at[pl.ds(base, C)], didxs[b])
            pltpu.sync_copy(e_h.at[pl.ds(base, C)], ebufs[b])
            pltpu.make_async_copy(h2t.at[sidxs[b]],
                                  rowss[b], sems[b]).start()

        def finish(b):
            pltpu.make_async_copy(h2t.at[sidxs[b]],
                                  rowss[b], sems[b]).wait()
            rows = rowss[b]
            ebuf = ebufs[b]

            def group(g, _):
                ev = ebuf[pl.ds(g * 16, 16)]
                for l in range(16):
                    e = jnp.full((16,), ev[l], jnp.float32)
                    r = g * 16 + l
                    for v in range(8):
                        rows[r, pl.ds(v * 16, 16)] = (
                            rows[r, pl.ds(v * 16, 16)] * e)
                return 0
            lax.fori_loop(0, G, group, 0)
            pltpu.sync_copy(rows, acc_sh.at[didxs[b]], add=True)

        start(0, 0)

        def pair(p, _):
            start(2 * p + 1, 1)
            finish(0)
            if nch % 2 == 1:
                start(2 * p + 2, 0)
            else:
                @pl.when(p < nch // 2 - 1)
                def _():
                    start(2 * p + 2, 0)
            finish(1)
            return 0
        lax.fori_loop(0, nch // 2, pair, 0)
        if nch % 2 == 1:
            finish(0)
        plsc.subcore_barrier()

        def wcp_core(num_hbm):
            def wcp(i, _):
                pltpu.sync_copy(acc_sh.at[pl.ds(zbase + i * 64, 64)], buf)

                def q16(q, _):
                    rc = rec_v[pl.ds(i * 64 + q * 16, 16)]
                    for l in range(16):
                        m = jnp.full((16,), rc[l], jnp.float32)
                        row = q * 16 + l
                        for v in range(8):
                            buf[row, pl.ds(v * 16, 16)] = (
                                buf[row, pl.ds(v * 16, 16)] * m)
                    return 0
                lax.fori_loop(0, 4, q16, 0)
                pltpu.sync_copy(buf, num_hbm.at[pl.ds(zbase + i * 64, 64)])
                return 0
            lax.fori_loop(0, nzc, wcp, 0)

        for cc in range(NC):
            @pl.when(cid == cc)
            def _():
                wcp_core([numa, numb][cc])

    return pl.kernel(
        body,
        out_type=[jax.ShapeDtypeStruct((npad, 128), jnp.float32)] * 2,
        mesh=mesh,
        compiler_params=_SC_PARAMS,
        scratch_types=[
            pltpu.VMEM_SHARED((npad, 128), jnp.float32),
            pltpu.VMEM((npad // NT,), jnp.float32),
            pltpu.VMEM((C,), jnp.float32),
            pltpu.VMEM((C,), jnp.float32),
            pltpu.VMEM((C,), jnp.int32),
            pltpu.VMEM((C,), jnp.int32),
            pltpu.VMEM((C,), jnp.int32),
            pltpu.VMEM((C,), jnp.int32),
            pltpu.VMEM((C, 128), jnp.float32),
            pltpu.VMEM((C, 128), jnp.float32),
            pltpu.VMEM((64, 128), jnp.float32),
            pltpu.SemaphoreType.DMA,
            pltpu.SemaphoreType.DMA,
        ],
    )


def _tc1_body(n0, n1, n2, n3, b1_ref, w2_ref, ms2_ref, md2_ref,
              h2_ref, tsf_ref, tdf_ref):
    h_mid = jnp.concatenate(
        [n0[...], n1[...], n2[...], n3[...]], axis=1) + b1_ref[...]
    h_mid = jnp.maximum(h_mid, 0.0)
    h2 = h_mid @ w2_ref[...]
    h2_ref[...] = h2
    tsf_ref[...] = h2 @ ms2_ref[...]
    tdf_ref[...] = h2 @ md2_ref[...]


def _tc1(nums, b1, W2, Ms2, Md2, npad):
    fb = pl.BlockSpec((BLK, 128), lambda i: (i, 0))
    return pl.pallas_call(
        _tc1_body,
        grid=(npad // BLK,),
        in_specs=[fb] * 4 + [
            pl.BlockSpec((1, 512), lambda i: (0, 0)),
            pl.BlockSpec((512, 128), lambda i: (0, 0)),
            pl.BlockSpec((128, 128), lambda i: (0, 0)),
            pl.BlockSpec((128, 128), lambda i: (0, 0)),
        ],
        out_specs=[fb, fb, fb],
        out_shape=[jax.ShapeDtypeStruct((npad, 128), jnp.float32)] * 3,
    )(*nums, b1.reshape(1, 512), W2, Ms2, Md2)


def _make_tc2_body(n_real, nblocks):
    def body(na, nb, b2_ref, p1_ref, pb1_ref, p2_ref, pb2_ref,
             zp_ref, gsum_ref, gp_ref):
        i = pl.program_id(0)
        z = na[...] + nb[...] + b2_ref[...]
        zp_ref[...] = jnp.maximum(z @ p1_ref[...] + pb1_ref[...],
                                  0.0) @ p2_ref[...] + pb2_ref[...]
        rid = BLK * i + lax.broadcasted_iota(jnp.int32, (BLK, 1), 0)
        part = jnp.sum(jnp.where(rid < n_real, z, 0.0), axis=0,
                       keepdims=True)

        @pl.when(i == 0)
        def _():
            gsum_ref[...] = part

        @pl.when(i > 0)
        def _():
            gsum_ref[...] = gsum_ref[...] + part

        @pl.when(i == nblocks - 1)
        def _():
            g = gsum_ref[...] / float(n_real)
            gp_ref[...] = jnp.maximum(g @ p1_ref[...] + pb1_ref[...],
                                      0.0) @ p2_ref[...] + pb2_ref[...]
    return body


def _tc2(numa, numb, b2, P1, pb1, P2, pb2, n_real, npad):
    nblocks = npad // BLK
    fb = pl.BlockSpec((BLK, 128), lambda i: (i, 0))
    one = pl.BlockSpec((1, 128), lambda i: (0, 0))
    return pl.pallas_call(
        _make_tc2_body(n_real, nblocks),
        grid=(nblocks,),
        in_specs=[fb, fb,
                  pl.BlockSpec((1, 128), lambda i: (0, 0)),
                  pl.BlockSpec((128, 64), lambda i: (0, 0)),
                  pl.BlockSpec((1, 64), lambda i: (0, 0)),
                  pl.BlockSpec((64, 128), lambda i: (0, 0)),
                  pl.BlockSpec((1, 128), lambda i: (0, 0))],
        out_specs=[fb, one, one],
        out_shape=[jax.ShapeDtypeStruct((npad, 128), jnp.float32),
                   jax.ShapeDtypeStruct((1, 128), jnp.float32),
                   jax.ShapeDtypeStruct((1, 128), jnp.float32)],
    )(numa, numb, b2.reshape(1, 128), P1, pb1.reshape(1, 64),
      P2, pb2.reshape(1, 128))


def _logit_mat(a, heads, ch):
    af = a.reshape(heads, ch)
    cols = []
    for c in range(128):
        h_ = (c // 16) % heads
        v = jnp.zeros((heads * ch,), jnp.float32)
        v = v.at[ch * h_:ch * (h_ + 1)].set(af[h_])
        cols.append(v)
    return jnp.stack(cols, axis=1)


def kernel(x, edge_index, W1, a_src1, a_dst1, b1, W2, a_src2, a_dst2, b2,
           P1, pb1, P2, pb2):
    N, _ = x.shape
    E = edge_index.shape[1]
    npad = ((N + (64 * NT) - 1) // (64 * NT)) * (64 * NT)
    gran = C * NT * NC
    epad = ((E + N + gran - 1) // gran) * gran

    xp = jnp.pad(x, ((0, npad - N), (0, 0)))
    sl = jnp.arange(N, dtype=jnp.int32)
    pad_n = epad - E - N
    src = jnp.concatenate(
        [edge_index[0].astype(jnp.int32), sl,
         jnp.zeros((pad_n,), jnp.int32)])
    dst = jnp.concatenate(
        [edge_index[1].astype(jnp.int32), sl,
         jnp.full((pad_n,), N, jnp.int32)])

    Ms = _logit_mat(a_src1, 8, 64)
    Md = _logit_mat(a_dst1, 8, 64)
    h1p0, h1p1, h1p2, h1p3, tsf, tdf = _tc0(xp, W1, Ms, Md, npad)

    tabs = []
    for j in range(4):
        t = jnp.stack([tsf[:, 32 * j], tsf[:, 32 * j + 16],
                       tdf[:, 32 * j], tdf[:, 32 * j + 16]], axis=1)
        tabs.append(t.reshape(-1))

    a1 = _make_sc_alpha1(npad, epad)(src, dst, *tabs)
    evals1, dens1 = a1[0:4], a1[4:8]
    recs1 = _tcr1(dens1, npad)
    recs1 = [r.reshape(-1) for r in recs1]

    sc_agg1 = _make_sc_agg1(npad, epad)
    nums = sc_agg1(src, dst, h1p0, h1p1, h1p2, h1p3, *evals1, *recs1)

    Ms2 = jnp.tile(a_src2.reshape(128, 1), (1, 128))
    Md2 = jnp.tile(a_dst2.reshape(128, 1), (1, 128))
    h2, tsf2, tdf2 = _tc1(nums, b1, W2, Ms2, Md2, npad)

    a2 = _make_sc_alpha2(npad, epad)(src, dst, tsf2[:, 0], tdf2[:, 0])
    evals2, dens2 = a2
    rec2 = _tcr2(dens2, npad).reshape(-1)

    numa, numb = _make_sc_agg2(npad, epad)(src, dst, h2, evals2, rec2)

    zp, _, gp = _tc2(numa, numb, b2, P1, pb1, P2, pb2, N, npad)
    return (zp[:N], gp)


# speedup vs baseline: 22.8537x; 1.0042x over previous
import jax
import jax.numpy as jnp
from jax import lax
from jax.experimental import pallas as pl
from jax.experimental.pallas import tpu as pltpu
from jax.experimental.pallas import tpu_sc as plsc

C = 128
G = C // 16
NT = 16
NC = 2
BLK = 256

_SC_PARAMS = pltpu.CompilerParams(needs_layout_passes=False)


def _zero_1d(ref, n):
    def z(i, _):
        ref[pl.ds(i * 16, 16)] = jnp.zeros((16,), jnp.float32)
        return 0
    lax.fori_loop(0, n // 16, z, 0)


def _leaky_exp(v):
    return jnp.exp(jnp.where(v >= 0.0, v, 0.2 * v))


def _tc0_body(x_ref, w1_ref, ms_ref, md_ref, h0, h1, h2, h3, tsf, tdf):
    h = x_ref[...] @ w1_ref[...]
    for j, r in enumerate((h0, h1, h2, h3)):
        r[...] = h[:, j * 128:(j + 1) * 128]
    tsf[...] = h @ ms_ref[...]
    tdf[...] = h @ md_ref[...]


def _tc0(xp, W1, Ms, Md, npad):
    fb = pl.BlockSpec((BLK, 128), lambda i: (i, 0))
    return pl.pallas_call(
        _tc0_body,
        grid=(npad // BLK,),
        in_specs=[
            pl.BlockSpec((BLK, 128), lambda i: (i, 0)),
            pl.BlockSpec((128, 512), lambda i: (0, 0)),
            pl.BlockSpec((512, 128), lambda i: (0, 0)),
            pl.BlockSpec((512, 128), lambda i: (0, 0)),
        ],
        out_specs=[fb] * 6,
        out_shape=[jax.ShapeDtypeStruct((npad, 128), jnp.float32)] * 6,
    )(xp, W1, Ms, Md)


def _make_sc_alpha1(npad, epad):
    epw = epad // 8
    nch = epw // C
    mesh = plsc.VectorSubcoreMesh(core_axis_name="c", subcore_axis_name="s")

    def body(src_h, dst_h, t0, t1, t2, t3, e0_h, e1_h, e2_h, e3_h,
             r0_h, r1_h, r2_h, r3_h,
             tab_v, denp_v, ebuf, sidx, didx, densh):
        trefs = [t0, t1, t2, t3]
        erefs = [e0_h, e1_h, e2_h, e3_h]
        rrefs = [r0_h, r1_h, r2_h, r3_h]
        cid = lax.axis_index("c")
        sid = lax.axis_index("s")
        iota = lax.broadcasted_iota(jnp.int32, (16,), 0)
        seg = 2 * npad // NT

        def work(tab_hbm, e_hbm, w):
            pltpu.sync_copy(tab_hbm, tab_v)
            _zero_1d(denp_v, 2 * npad)

            def chunk(k, _):
                base = w * epw + k * C
                pltpu.sync_copy(src_h.at[pl.ds(base, C)], sidx)
                pltpu.sync_copy(dst_h.at[pl.ds(base, C)], didx)

                def group(g, _):
                    sv = sidx[pl.ds(g * 16, 16)]
                    dv = didx[pl.ds(g * 16, 16)]
                    a0 = plsc.load_gather(tab_v, [sv * 4])
                    a1 = plsc.load_gather(tab_v, [sv * 4 + 1])
                    b0 = plsc.load_gather(tab_v, [dv * 4 + 2])
                    b1 = plsc.load_gather(tab_v, [dv * 4 + 3])
                    e0v = _leaky_exp(a0 + b0)
                    e1v = _leaky_exp(a1 + b1)
                    ebuf[pl.ds(g * 16, 16)] = e0v
                    ebuf[pl.ds(C + g * 16, 16)] = e1v
                    for l in range(16):
                        e0b = jnp.full((16,), e0v[l], jnp.float32)
                        e1b = jnp.full((16,), e1v[l], jnp.float32)
                        di = (jnp.full((16,), dv[l], jnp.int32)
                              + (iota & 1) * npad)
                        vals = jnp.where(iota == 0, e0b, e1b)
                        plsc.addupdate_scatter(denp_v, [di], vals,
                                               mask=iota < 2)
                    return 0
                lax.fori_loop(0, G, group, 0)
                pltpu.sync_copy(ebuf, e_hbm.at[pl.ds(2 * base, 2 * C)])
                return 0
            lax.fori_loop(0, nch, chunk, 0)
            pltpu.sync_copy(denp_v,
                            densh.at[pl.ds(sid * 2 * npad, 2 * npad)])

        for cc in range(NC):
            @pl.when(cid == cc)
            def _():
                for half in range(2):
                    hp = cc * 2 + half
                    pred = (sid < 8) if half == 0 else (sid >= 8)

                    @pl.when(pred)
                    def _():
                        work(trefs[hp], erefs[hp], sid - half * 8)

                plsc.subcore_barrier()
                off = sid * seg
                for half in range(2):
                    rbase = half * 8
                    for r in range(8):
                        pltpu.sync_copy(
                            densh.at[pl.ds((rbase + r) * 2 * npad + off,
                                           seg)],
                            denp_v.at[pl.ds(r * seg, seg)])

                    def red(j, _):
                        acc = denp_v[pl.ds(j * 16, 16)]
                        for r in range(1, 8):
                            acc = acc + denp_v[pl.ds(r * seg + j * 16,
                                                     16)]
                        denp_v[pl.ds(j * 16, 16)] = 1.0 / (acc + 1e-16)
                        return 0
                    lax.fori_loop(0, seg // 16, red, 0)
                    pltpu.sync_copy(
                        denp_v.at[pl.ds(0, seg)],
                        rrefs[cc * 2 + half].at[pl.ds(off, seg)])

    return pl.kernel(
        body,
        out_type=[jax.ShapeDtypeStruct((2 * epad,), jnp.float32)] * 4
        + [jax.ShapeDtypeStruct((2 * npad,), jnp.float32)] * 4,
        mesh=mesh,
        compiler_params=_SC_PARAMS,
        scratch_types=[
            pltpu.VMEM((4 * npad,), jnp.float32),
            pltpu.VMEM((2 * npad,), jnp.float32),
            pltpu.VMEM((2 * C,), jnp.float32),
            pltpu.VMEM((C,), jnp.int32),
            pltpu.VMEM((C,), jnp.int32),
            pltpu.VMEM_SHARED((NT * 2 * npad,), jnp.float32),
        ],
    )


def _tcr1_body(d0, d1, d2, d3, r0, r1, r2, r3):
    for dref, rref in zip((d0, d1, d2, d3), (r0, r1, r2, r3)):
        s = jnp.sum(dref[...], axis=0)
        rref[...] = 1.0 / (s + 1e-16)


def _tcr1(dens, npad):
    db = pl.BlockSpec((8, 2, BLK), lambda i: (0, 0, i))
    rb = pl.BlockSpec((2, BLK), lambda i: (0, i))
    return pl.pallas_call(
        _tcr1_body,
        grid=(npad // BLK,),
        in_specs=[db] * 4,
        out_specs=[rb] * 4,
        out_shape=[jax.ShapeDtypeStruct((2, npad), jnp.float32)] * 4,
    )(*[d.reshape(8, 2, npad) for d in dens])


def _make_sc_agg1(npad, epad):
    rpt = npad // NT
    nzc = rpt // 64
    nch = epad // C // NT
    mesh = plsc.VectorSubcoreMesh(core_axis_name="c", subcore_axis_name="s")

    def body(src_h, dst_h, h0, h1, h2, h3, e0_h, e1_h, e2_h, e3_h,
             r0_h, r1_h, r2_h, r3_h, num0, num1, num2, num3,
             acc_sh, rec0_v, rec1_v, ebuf_a, ebuf_b, sidx_a, sidx_b,
             didx_a, didx_b, rows_a, rows_b, buf, sem_a, sem_b):
        hrefs = [h0, h1, h2, h3]
        erefs = [e0_h, e1_h, e2_h, e3_h]
        rrefs = [r0_h, r1_h, r2_h, r3_h]
        numrefs = [num0, num1, num2, num3]
        cid = lax.axis_index("c")
        sid = lax.axis_index("s")
        zbase = sid * rpt
        sems = [sem_a, sem_b]
        ebufs = [ebuf_a, ebuf_b]
        sidxs = [sidx_a, sidx_b]
        didxs = [didx_a, didx_b]
        rowss = [rows_a, rows_b]

        def job(rows_hbm, e_hbm, rec_hbm, num_hbm):
            pltpu.sync_copy(rec_hbm.at[pl.ds(sid * rpt, rpt)], rec0_v)
            pltpu.sync_copy(rec_hbm.at[pl.ds(npad + sid * rpt, rpt)],
                            rec1_v)
            def zcp(i, _):
                pltpu.sync_copy(buf, acc_sh.at[pl.ds(zbase + i * 64, 64)])
                return 0

            def zrow(i, _):
                for v in range(8):
                    buf[i, pl.ds(v * 16, 16)] = jnp.zeros((16,),
                                                          jnp.float32)
                return 0
            lax.fori_loop(0, 64, zrow, 0)
            lax.fori_loop(0, nzc, zcp, 0)
            plsc.subcore_barrier()

            def start(k, b):
                base = (sid * nch + k) * C
                pltpu.sync_copy(src_h.at[pl.ds(base, C)], sidxs[b])
                pltpu.sync_copy(dst_h.at[pl.ds(base, C)], didxs[b])
                pltpu.sync_copy(e_hbm.at[pl.ds(2 * base, 2 * C)],
                                ebufs[b])
                pltpu.make_async_copy(rows_hbm.at[sidxs[b]],
                                      rowss[b], sems[b]).start()

            def finish(b):
                pltpu.make_async_copy(rows_hbm.at[sidxs[b]],
                                      rowss[b], sems[b]).wait()
                rows = rowss[b]
                ebuf = ebufs[b]

                def group(g, _):
                    e0v = ebuf[pl.ds(g * 16, 16)]
                    e1v = ebuf[pl.ds(C + g * 16, 16)]
                    for l in range(16):
                        e0 = jnp.full((16,), e0v[l], jnp.float32)
                        e1 = jnp.full((16,), e1v[l], jnp.float32)
                        r = g * 16 + l
                        for v in range(8):
                            m = e0 if v < 4 else e1
                            rows[r, pl.ds(v * 16, 16)] = (
                                rows[r, pl.ds(v * 16, 16)] * m)
                    return 0
                lax.fori_loop(0, G, group, 0)
                pltpu.sync_copy(rows, acc_sh.at[didxs[b]], add=True)

            start(0, 0)

            def pair(p, _):
                start(2 * p + 1, 1)
                finish(0)

                @pl.when(p < nch // 2 - 1)
                def _():
                    start(2 * p + 2, 0)
                finish(1)
                return 0
            lax.fori_loop(0, nch // 2, pair, 0)
            plsc.subcore_barrier()

            def wcp(i, _):
                pltpu.sync_copy(acc_sh.at[pl.ds(zbase + i * 64, 64)], buf)

                def q8(q, _):
                    rc0 = rec0_v[pl.ds(i * 64 + q * 16, 16)]
                    rc1 = rec1_v[pl.ds(i * 64 + q * 16, 16)]
                    for l in range(16):
                        m0 = jnp.full((16,), rc0[l], jnp.float32)
                        m1 = jnp.full((16,), rc1[l], jnp.float32)
                        row = q * 16 + l
                        for v in range(8):
                            m = m0 if v < 4 else m1
                            buf[row, pl.ds(v * 16, 16)] = (
                                buf[row, pl.ds(v * 16, 16)] * m)
                    return 0
                lax.fori_loop(0, 4, q8, 0)
                pltpu.sync_copy(buf, num_hbm.at[pl.ds(zbase + i * 64, 64)])
                return 0
            lax.fori_loop(0, nzc, wcp, 0)
            plsc.subcore_barrier()

        for cc in range(NC):
            @pl.when(cid == cc)
            def _():
                for jj in range(2):
                    hp = cc * 2 + jj
                    job(hrefs[hp], erefs[hp], rrefs[hp], numrefs[hp])

    return pl.kernel(
        body,
        out_type=[jax.ShapeDtypeStruct((npad, 128), jnp.float32)] * 4,
        mesh=mesh,
        compiler_params=_SC_PARAMS,
        scratch_types=[
            pltpu.VMEM_SHARED((npad, 128), jnp.float32),
            pltpu.VMEM((npad // NT,), jnp.float32),
            pltpu.VMEM((npad // NT,), jnp.float32),
            pltpu.VMEM((2 * C,), jnp.float32),
            pltpu.VMEM((2 * C,), jnp.float32),
            pltpu.VMEM((C,), jnp.int32),
            pltpu.VMEM((C,), jnp.int32),
            pltpu.VMEM((C,), jnp.int32),
            pltpu.VMEM((C,), jnp.int32),
            pltpu.VMEM((C, 128), jnp.float32),
            pltpu.VMEM((C, 128), jnp.float32),
            pltpu.VMEM((64, 128), jnp.float32),
            pltpu.SemaphoreType.DMA,
            pltpu.SemaphoreType.DMA,
        ],
    )


def _make_sc_alpha2(npad, epad):
    epw = epad // (NT * NC)
    nch = epw // C
    mesh = plsc.VectorSubcoreMesh(core_axis_name="c", subcore_axis_name="s")

    def body(src_h, dst_h, ts_h, td_h, e_h, dena_h, denb_h,
             ts_v, td_v, denp_v, ebuf, sidx, didx, densh):
        cid = lax.axis_index("c")
        sid = lax.axis_index("s")
        wid = cid * NT + sid
        iota = lax.broadcasted_iota(jnp.int32, (16,), 0)
        seg = npad // NT
        pltpu.sync_copy(ts_h, ts_v)
        pltpu.sync_copy(td_h, td_v)
        _zero_1d(denp_v, npad)

        def chunk(k, _):
            base = (wid * nch + k) * C
            pltpu.sync_copy(src_h.at[pl.ds(base, C)], sidx)
            pltpu.sync_copy(dst_h.at[pl.ds(base, C)], didx)

            def group(g, _):
                sv = sidx[pl.ds(g * 16, 16)]
                dv = didx[pl.ds(g * 16, 16)]
                a = plsc.load_gather(ts_v, [sv])
                b = plsc.load_gather(td_v, [dv])
                ev = _leaky_exp(a + b)
                ebuf[pl.ds(g * 16, 16)] = ev
                for l in range(16):
                    eb = jnp.full((16,), ev[l], jnp.float32)
                    di = jnp.full((16,), dv[l], jnp.int32)
                    plsc.addupdate_scatter(denp_v, [di], eb,
                                           mask=iota < 1)
                return 0
            lax.fori_loop(0, G, group, 0)
            pltpu.sync_copy(ebuf, e_h.at[pl.ds(base, C)])
            return 0
        lax.fori_loop(0, nch, chunk, 0)
        pltpu.sync_copy(denp_v, densh.at[pl.ds(sid * npad, npad)])
        plsc.subcore_barrier()

        off = sid * seg
        for r in range(NT):
            pltpu.sync_copy(densh.at[pl.ds(r * npad + off, seg)],
                            denp_v.at[pl.ds(r * seg, seg)])

        def red(j, _):
            acc = denp_v[pl.ds(j * 16, 16)]
            for r in range(1, NT):
                acc = acc + denp_v[pl.ds(r * seg + j * 16, 16)]
            denp_v[pl.ds(j * 16, 16)] = acc
            return 0
        lax.fori_loop(0, seg // 16, red, 0)
        for cc in range(NC):
            @pl.when(cid == cc)
            def _():
                pltpu.sync_copy(denp_v.at[pl.ds(0, seg)],
                                [dena_h, denb_h][cc].at[pl.ds(off, seg)])

    return pl.kernel(
        body,
        out_type=[jax.ShapeDtypeStruct((epad,), jnp.float32),
                  jax.ShapeDtypeStruct((npad,), jnp.float32),
                  jax.ShapeDtypeStruct((npad,), jnp.float32)],
        mesh=mesh,
        compiler_params=_SC_PARAMS,
        scratch_types=[
            pltpu.VMEM((npad,), jnp.float32),
            pltpu.VMEM((npad,), jnp.float32),
            pltpu.VMEM((npad,), jnp.float32),
            pltpu.VMEM((C,), jnp.float32),
            pltpu.VMEM((C,), jnp.int32),
            pltpu.VMEM((C,), jnp.int32),
            pltpu.VMEM_SHARED((NT * npad,), jnp.float32),
        ],
    )


def _tcr2_body(d_ref, r_ref):
    s = jnp.sum(d_ref[...], axis=0)
    r_ref[...] = (1.0 / (s + 1e-16)).reshape(1, BLK)


def _tcr2(den, npad):
    return pl.pallas_call(
        _tcr2_body,
        grid=(npad // BLK,),
        in_specs=[pl.BlockSpec((NT * NC, BLK), lambda i: (0, i))],
        out_specs=pl.BlockSpec((1, BLK), lambda i: (0, i)),
        out_shape=jax.ShapeDtypeStruct((1, npad), jnp.float32),
    )(den.reshape(NT * NC, npad))


def _make_sc_agg2(npad, epad):
    rpt = npad // NT
    nzc = rpt // 64
    nch = epad // C // (NT * NC)
    mesh = plsc.VectorSubcoreMesh(core_axis_name="c", subcore_axis_name="s")

    def body(src_h, dst_h, h2t, e_h, dena_h, denb_h, numa, numb,
             acc_sh, rec_v, tmp_v, ebuf_a, ebuf_b, sidx_a, sidx_b,
             didx_a, didx_b, rows_a, rows_b, buf,
             sem_a, sem_b):
        cid = lax.axis_index("c")
        sid = lax.axis_index("s")
        zbase = sid * rpt
        wid = cid * NT + sid
        sems = [sem_a, sem_b]
        ebufs = [ebuf_a, ebuf_b]
        sidxs = [sidx_a, sidx_b]
        didxs = [didx_a, didx_b]
        rowss = [rows_a, rows_b]
        pltpu.sync_copy(dena_h.at[pl.ds(sid * rpt, rpt)], rec_v)
        pltpu.sync_copy(denb_h.at[pl.ds(sid * rpt, rpt)], tmp_v)

        def mkrec(j, _):
            a = rec_v[pl.ds(j * 16, 16)]
            b = tmp_v[pl.ds(j * 16, 16)]
            rec_v[pl.ds(j * 16, 16)] = 1.0 / (a + b + 1e-16)
            return 0
        lax.fori_loop(0, rpt // 16, mkrec, 0)

        def zrow(i, _):
            for v in range(8):
                buf[i, pl.ds(v * 16, 16)] = jnp.zeros((16,), jnp.float32)
            return 0
        lax.fori_loop(0, 64, zrow, 0)

        def zcp(i, _):
            pltpu.sync_copy(buf, acc_sh.at[pl.ds(zbase + i * 64, 64)])
            return 0
        lax.fori_loop(0, nzc, zcp, 0)
        plsc.subcore_barrier()

        def start(k, b):
            base = (wid * nch + k) * C
            pltpu.sync_copy(src_h.at[pl.ds(base, C)], sidxs[b])
            pltpu.sync_copy(dst_h.at[pl.ds(base, C)], didxs[b])
            pltpu.sync_copy(e_h.at[pl.ds(base, C)], ebufs[b])
            pltpu.make_async_copy(h2t.at[sidxs[b]],
                                  rowss[b], sems[b]).start()

        def finish(b):
            pltpu.make_async_copy(h2t.at[sidxs[b]],
                                  rowss[b], sems[b]).wait()
            rows = rowss[b]
            ebuf = ebufs[b]

            def group(g, _):
                ev = ebuf[pl.ds(g * 16, 16)]
                for l in range(16):
                    e = jnp.full((16,), ev[l], jnp.float32)
                    r = g * 16 + l
                    for v in range(8):
                        rows[r, pl.ds(v * 16, 16)] = (
                            rows[r, pl.ds(v * 16, 16)] * e)
                return 0
            lax.fori_loop(0, G, group, 0)
            pltpu.sync_copy(rows, acc_sh.at[didxs[b]], add=True)

        start(0, 0)

        def pair(p, _):
            start(2 * p + 1, 1)
            finish(0)
            if nch % 2 == 1:
                start(2 * p + 2, 0)
            else:
                @pl.when(p < nch // 2 - 1)
                def _():
                    start(2 * p + 2, 0)
            finish(1)
            return 0
        lax.fori_loop(0, nch // 2, pair, 0)
        if nch % 2 == 1:
            finish(0)
        plsc.subcore_barrier()

        def wcp_core(num_hbm):
            def wcp(i, _):
                pltpu.sync_copy(acc_sh.at[pl.ds(zbase + i * 64, 64)], buf)

                def q16(q, _):
                    rc = rec_v[pl.ds(i * 64 + q * 16, 16)]
                    for l in range(16):
                        m = jnp.full((16,), rc[l], jnp.float32)
                        row = q * 16 + l
                        for v in range(8):
                            buf[row, pl.ds(v * 16, 16)] = (
                                buf[row, pl.ds(v * 16, 16)] * m)
                    return 0
                lax.fori_loop(0, 4, q16, 0)
                pltpu.sync_copy(buf, num_hbm.at[pl.ds(zbase + i * 64, 64)])
                return 0
            lax.fori_loop(0, nzc, wcp, 0)

        for cc in range(NC):
            @pl.when(cid == cc)
            def _():
                wcp_core([numa, numb][cc])

    return pl.kernel(
        body,
        out_type=[jax.ShapeDtypeStruct((npad, 128), jnp.float32)] * 2,
        mesh=mesh,
        compiler_params=_SC_PARAMS,
        scratch_types=[
            pltpu.VMEM_SHARED((npad, 128), jnp.float32),
            pltpu.VMEM((npad // NT,), jnp.float32),
            pltpu.VMEM((npad // NT,), jnp.float32),
            pltpu.VMEM((C,), jnp.float32),
            pltpu.VMEM((C,), jnp.float32),
            pltpu.VMEM((C,), jnp.int32),
            pltpu.VMEM((C,), jnp.int32),
            pltpu.VMEM((C,), jnp.int32),
            pltpu.VMEM((C,), jnp.int32),
            pltpu.VMEM((C, 128), jnp.float32),
            pltpu.VMEM((C, 128), jnp.float32),
            pltpu.VMEM((64, 128), jnp.float32),
            pltpu.SemaphoreType.DMA,
            pltpu.SemaphoreType.DMA,
        ],
    )


def _tc1_body(n0, n1, n2, n3, b1_ref, w2_ref, ms2_ref, md2_ref,
              h2_ref, tsf_ref, tdf_ref):
    h_mid = jnp.concatenate(
        [n0[...], n1[...], n2[...], n3[...]], axis=1) + b1_ref[...]
    h_mid = jnp.maximum(h_mid, 0.0)
    h2 = h_mid @ w2_ref[...]
    h2_ref[...] = h2
    tsf_ref[...] = h2 @ ms2_ref[...]
    tdf_ref[...] = h2 @ md2_ref[...]


def _tc1(nums, b1, W2, Ms2, Md2, npad):
    fb = pl.BlockSpec((BLK, 128), lambda i: (i, 0))
    return pl.pallas_call(
        _tc1_body,
        grid=(npad // BLK,),
        in_specs=[fb] * 4 + [
            pl.BlockSpec((1, 512), lambda i: (0, 0)),
            pl.BlockSpec((512, 128), lambda i: (0, 0)),
            pl.BlockSpec((128, 128), lambda i: (0, 0)),
            pl.BlockSpec((128, 128), lambda i: (0, 0)),
        ],
        out_specs=[fb, fb, fb],
        out_shape=[jax.ShapeDtypeStruct((npad, 128), jnp.float32)] * 3,
    )(*nums, b1.reshape(1, 512), W2, Ms2, Md2)


def _make_tc2_body(n_real, nblocks):
    def body(na, nb, b2_ref, p1_ref, pb1_ref, p2_ref, pb2_ref,
             zp_ref, gsum_ref, gp_ref):
        i = pl.program_id(0)
        z = na[...] + nb[...] + b2_ref[...]
        zp_ref[...] = jnp.maximum(z @ p1_ref[...] + pb1_ref[...],
                                  0.0) @ p2_ref[...] + pb2_ref[...]
        rid = BLK * i + lax.broadcasted_iota(jnp.int32, (BLK, 1), 0)
        part = jnp.sum(jnp.where(rid < n_real, z, 0.0), axis=0,
                       keepdims=True)

        @pl.when(i == 0)
        def _():
            gsum_ref[...] = part

        @pl.when(i > 0)
        def _():
            gsum_ref[...] = gsum_ref[...] + part

        @pl.when(i == nblocks - 1)
        def _():
            g = gsum_ref[...] / float(n_real)
            gp_ref[...] = jnp.maximum(g @ p1_ref[...] + pb1_ref[...],
                                      0.0) @ p2_ref[...] + pb2_ref[...]
    return body


def _tc2(numa, numb, b2, P1, pb1, P2, pb2, n_real, npad):
    nblocks = npad // BLK
    fb = pl.BlockSpec((BLK, 128), lambda i: (i, 0))
    one = pl.BlockSpec((1, 128), lambda i: (0, 0))
    return pl.pallas_call(
        _make_tc2_body(n_real, nblocks),
        grid=(nblocks,),
        in_specs=[fb, fb,
                  pl.BlockSpec((1, 128), lambda i: (0, 0)),
                  pl.BlockSpec((128, 64), lambda i: (0, 0)),
                  pl.BlockSpec((1, 64), lambda i: (0, 0)),
                  pl.BlockSpec((64, 128), lambda i: (0, 0)),
                  pl.BlockSpec((1, 128), lambda i: (0, 0))],
        out_specs=[fb, one, one],
        out_shape=[jax.ShapeDtypeStruct((npad, 128), jnp.float32),
                   jax.ShapeDtypeStruct((1, 128), jnp.float32),
                   jax.ShapeDtypeStruct((1, 128), jnp.float32)],
    )(numa, numb, b2.reshape(1, 128), P1, pb1.reshape(1, 64),
      P2, pb2.reshape(1, 128))


def _logit_mat(a, heads, ch):
    af = a.reshape(heads, ch)
    cols = []
    for c in range(128):
        h_ = (c // 16) % heads
        v = jnp.zeros((heads * ch,), jnp.float32)
        v = v.at[ch * h_:ch * (h_ + 1)].set(af[h_])
        cols.append(v)
    return jnp.stack(cols, axis=1)


def kernel(x, edge_index, W1, a_src1, a_dst1, b1, W2, a_src2, a_dst2, b2,
           P1, pb1, P2, pb2):
    N, _ = x.shape
    E = edge_index.shape[1]
    npad = ((N + (64 * NT) - 1) // (64 * NT)) * (64 * NT)
    gran = C * NT * NC
    epad = ((E + N + gran - 1) // gran) * gran

    xp = jnp.pad(x, ((0, npad - N), (0, 0)))
    sl = jnp.arange(N, dtype=jnp.int32)
    pad_n = epad - E - N
    src = jnp.concatenate(
        [edge_index[0].astype(jnp.int32), sl,
         jnp.zeros((pad_n,), jnp.int32)])
    dst = jnp.concatenate(
        [edge_index[1].astype(jnp.int32), sl,
         jnp.full((pad_n,), N, jnp.int32)])

    Ms = _logit_mat(a_src1, 8, 64)
    Md = _logit_mat(a_dst1, 8, 64)
    h1p0, h1p1, h1p2, h1p3, tsf, tdf = _tc0(xp, W1, Ms, Md, npad)

    tabs = []
    for j in range(4):
        t = jnp.stack([tsf[:, 32 * j], tsf[:, 32 * j + 16],
                       tdf[:, 32 * j], tdf[:, 32 * j + 16]], axis=1)
        tabs.append(t.reshape(-1))

    a1 = _make_sc_alpha1(npad, epad)(src, dst, *tabs)
    evals1, recs1 = a1[0:4], a1[4:8]

    sc_agg1 = _make_sc_agg1(npad, epad)
    nums = sc_agg1(src, dst, h1p0, h1p1, h1p2, h1p3, *evals1, *recs1)

    Ms2 = jnp.tile(a_src2.reshape(128, 1), (1, 128))
    Md2 = jnp.tile(a_dst2.reshape(128, 1), (1, 128))
    h2, tsf2, tdf2 = _tc1(nums, b1, W2, Ms2, Md2, npad)

    a2 = _make_sc_alpha2(npad, epad)(src, dst, tsf2[:, 0], tdf2[:, 0])
    evals2, dena2, denb2 = a2

    numa, numb = _make_sc_agg2(npad, epad)(src, dst, h2, evals2,
                                           dena2, denb2)

    zp, _, gp = _tc2(numa, numb, b2, P1, pb1, P2, pb2, N, npad)
    return (zp[:N], gp)


# speedup vs baseline: 26.0013x; 1.1377x over previous
import jax
import jax.numpy as jnp
from jax import lax
from jax.experimental import pallas as pl
from jax.experimental.pallas import tpu as pltpu
from jax.experimental.pallas import tpu_sc as plsc

C = 128
G = C // 16
NT = 16
NC = 2
BLK = 256

_SC_PARAMS = pltpu.CompilerParams(needs_layout_passes=False)


def _zero_1d(ref, n):
    def z(i, _):
        ref[pl.ds(i * 16, 16)] = jnp.zeros((16,), jnp.float32)
        return 0
    lax.fori_loop(0, n // 16, z, 0)


def _leaky_exp(v):
    return jnp.exp(jnp.where(v >= 0.0, v, 0.2 * v))


def _tc0_body(x_ref, w1_ref, ms_ref, md_ref, h0, h1, h2, h3, tsf, tdf):
    h = x_ref[...] @ w1_ref[...]
    for j, r in enumerate((h0, h1, h2, h3)):
        r[...] = h[:, j * 128:(j + 1) * 128]
    tsf[...] = h @ ms_ref[...]
    tdf[...] = h @ md_ref[...]


def _tc0(xp, W1, Ms, Md, npad):
    fb = pl.BlockSpec((BLK, 128), lambda i: (i, 0))
    return pl.pallas_call(
        _tc0_body,
        grid=(npad // BLK,),
        in_specs=[
            pl.BlockSpec((BLK, 128), lambda i: (i, 0)),
            pl.BlockSpec((128, 512), lambda i: (0, 0)),
            pl.BlockSpec((512, 128), lambda i: (0, 0)),
            pl.BlockSpec((512, 128), lambda i: (0, 0)),
        ],
        out_specs=[fb] * 6,
        out_shape=[jax.ShapeDtypeStruct((npad, 128), jnp.float32)] * 6,
    )(xp, W1, Ms, Md)


def _make_sc_alpha1(npad, epad):
    epw = epad // 8
    CA = 512
    GA = CA // 16
    nch = epw // CA
    mesh = plsc.VectorSubcoreMesh(core_axis_name="c", subcore_axis_name="s")

    def body(src_h, dst_h, t0, t1, t2, t3, e0_h, e1_h, e2_h, e3_h,
             r0_h, r1_h, r2_h, r3_h,
             tab_v, denp_v, ebuf, sidx, didx, densh):
        trefs = [t0, t1, t2, t3]
        erefs = [e0_h, e1_h, e2_h, e3_h]
        rrefs = [r0_h, r1_h, r2_h, r3_h]
        cid = lax.axis_index("c")
        sid = lax.axis_index("s")
        iota = lax.broadcasted_iota(jnp.int32, (16,), 0)
        seg = 2 * npad // NT

        def work(tab_hbm, e_hbm, w):
            pltpu.sync_copy(tab_hbm, tab_v)
            _zero_1d(denp_v, 2 * npad)

            def chunk(k, _):
                base = w * epw + k * CA
                pltpu.sync_copy(src_h.at[pl.ds(base, CA)], sidx)
                pltpu.sync_copy(dst_h.at[pl.ds(base, CA)], didx)

                def group(g, _):
                    sv = sidx[pl.ds(g * 16, 16)]
                    dv = didx[pl.ds(g * 16, 16)]
                    a0 = plsc.load_gather(tab_v, [sv * 4])
                    a1 = plsc.load_gather(tab_v, [sv * 4 + 1])
                    b0 = plsc.load_gather(tab_v, [dv * 4 + 2])
                    b1 = plsc.load_gather(tab_v, [dv * 4 + 3])
                    e0v = _leaky_exp(a0 + b0)
                    e1v = _leaky_exp(a1 + b1)
                    eo = (g // 8) * 256 + (g % 8) * 16
                    ebuf[pl.ds(eo, 16)] = e0v
                    ebuf[pl.ds(eo + 128, 16)] = e1v
                    for l in range(16):
                        e0b = jnp.full((16,), e0v[l], jnp.float32)
                        e1b = jnp.full((16,), e1v[l], jnp.float32)
                        di = (jnp.full((16,), dv[l], jnp.int32)
                              + (iota & 1) * npad)
                        vals = jnp.where(iota == 0, e0b, e1b)
                        plsc.addupdate_scatter(denp_v, [di], vals,
                                               mask=iota < 2)
                    return 0
                lax.fori_loop(0, GA, group, 0)
                pltpu.sync_copy(ebuf, e_hbm.at[pl.ds(2 * base, 2 * CA)])
                return 0
            lax.fori_loop(0, nch, chunk, 0)
            pltpu.sync_copy(denp_v,
                            densh.at[pl.ds(sid * 2 * npad, 2 * npad)])

        for cc in range(NC):
            @pl.when(cid == cc)
            def _():
                for half in range(2):
                    hp = cc * 2 + half
                    pred = (sid < 8) if half == 0 else (sid >= 8)

                    @pl.when(pred)
                    def _():
                        work(trefs[hp], erefs[hp], sid - half * 8)

                plsc.subcore_barrier()
                off = sid * seg
                for half in range(2):
                    rbase = half * 8
                    for r in range(8):
                        pltpu.sync_copy(
                            densh.at[pl.ds((rbase + r) * 2 * npad + off,
                                           seg)],
                            denp_v.at[pl.ds(r * seg, seg)])

                    def red(j, _):
                        acc = denp_v[pl.ds(j * 16, 16)]
                        for r in range(1, 8):
                            acc = acc + denp_v[pl.ds(r * seg + j * 16,
                                                     16)]
                        denp_v[pl.ds(j * 16, 16)] = 1.0 / (acc + 1e-16)
                        return 0
                    lax.fori_loop(0, seg // 16, red, 0)
                    pltpu.sync_copy(
                        denp_v.at[pl.ds(0, seg)],
                        rrefs[cc * 2 + half].at[pl.ds(off, seg)])

    return pl.kernel(
        body,
        out_type=[jax.ShapeDtypeStruct((2 * epad,), jnp.float32)] * 4
        + [jax.ShapeDtypeStruct((2 * npad,), jnp.float32)] * 4,
        mesh=mesh,
        compiler_params=_SC_PARAMS,
        scratch_types=[
            pltpu.VMEM((4 * npad,), jnp.float32),
            pltpu.VMEM((2 * npad,), jnp.float32),
            pltpu.VMEM((2 * 512,), jnp.float32),
            pltpu.VMEM((512,), jnp.int32),
            pltpu.VMEM((512,), jnp.int32),
            pltpu.VMEM_SHARED((NT * 2 * npad,), jnp.float32),
        ],
    )


def _tcr1_body(d0, d1, d2, d3, r0, r1, r2, r3):
    for dref, rref in zip((d0, d1, d2, d3), (r0, r1, r2, r3)):
        s = jnp.sum(dref[...], axis=0)
        rref[...] = 1.0 / (s + 1e-16)


def _tcr1(dens, npad):
    db = pl.BlockSpec((8, 2, BLK), lambda i: (0, 0, i))
    rb = pl.BlockSpec((2, BLK), lambda i: (0, i))
    return pl.pallas_call(
        _tcr1_body,
        grid=(npad // BLK,),
        in_specs=[db] * 4,
        out_specs=[rb] * 4,
        out_shape=[jax.ShapeDtypeStruct((2, npad), jnp.float32)] * 4,
    )(*[d.reshape(8, 2, npad) for d in dens])


def _make_sc_agg1(npad, epad):
    rpt = npad // NT
    nzc = rpt // 64
    nch = epad // C // NT
    mesh = plsc.VectorSubcoreMesh(core_axis_name="c", subcore_axis_name="s")

    def body(src_h, dst_h, h0, h1, h2, h3, e0_h, e1_h, e2_h, e3_h,
             r0_h, r1_h, r2_h, r3_h, num0, num1, num2, num3,
             acc_sh, rec0_v, rec1_v, ebuf_a, ebuf_b, sidx_a, sidx_b,
             didx_a, didx_b, rows_a, rows_b, buf, sem_a, sem_b):
        hrefs = [h0, h1, h2, h3]
        erefs = [e0_h, e1_h, e2_h, e3_h]
        rrefs = [r0_h, r1_h, r2_h, r3_h]
        numrefs = [num0, num1, num2, num3]
        cid = lax.axis_index("c")
        sid = lax.axis_index("s")
        zbase = sid * rpt
        sems = [sem_a, sem_b]
        ebufs = [ebuf_a, ebuf_b]
        sidxs = [sidx_a, sidx_b]
        didxs = [didx_a, didx_b]
        rowss = [rows_a, rows_b]

        def job(rows_hbm, e_hbm, rec_hbm, num_hbm):
            pltpu.sync_copy(rec_hbm.at[pl.ds(sid * rpt, rpt)], rec0_v)
            pltpu.sync_copy(rec_hbm.at[pl.ds(npad + sid * rpt, rpt)],
                            rec1_v)
            def zcp(i, _):
                pltpu.sync_copy(buf, acc_sh.at[pl.ds(zbase + i * 64, 64)])
                return 0

            def zrow(i, _):
                for v in range(8):
                    buf[i, pl.ds(v * 16, 16)] = jnp.zeros((16,),
                                                          jnp.float32)
                return 0
            lax.fori_loop(0, 64, zrow, 0)
            lax.fori_loop(0, nzc, zcp, 0)
            plsc.subcore_barrier()

            def start(k, b):
                base = (sid * nch + k) * C
                pltpu.sync_copy(src_h.at[pl.ds(base, C)], sidxs[b])
                pltpu.sync_copy(dst_h.at[pl.ds(base, C)], didxs[b])
                pltpu.sync_copy(e_hbm.at[pl.ds(2 * base, 2 * C)],
                                ebufs[b])
                pltpu.make_async_copy(rows_hbm.at[sidxs[b]],
                                      rowss[b], sems[b]).start()

            def finish(b):
                pltpu.make_async_copy(rows_hbm.at[sidxs[b]],
                                      rowss[b], sems[b]).wait()
                rows = rowss[b]
                ebuf = ebufs[b]

                def group(g, _):
                    e0v = ebuf[pl.ds(g * 16, 16)]
                    e1v = ebuf[pl.ds(C + g * 16, 16)]
                    for l in range(16):
                        e0 = jnp.full((16,), e0v[l], jnp.float32)
                        e1 = jnp.full((16,), e1v[l], jnp.float32)
                        r = g * 16 + l
                        for v in range(8):
                            m = e0 if v < 4 else e1
                            rows[r, pl.ds(v * 16, 16)] = (
                                rows[r, pl.ds(v * 16, 16)] * m)
                    return 0
                lax.fori_loop(0, G, group, 0)
                pltpu.sync_copy(rows, acc_sh.at[didxs[b]], add=True)

            start(0, 0)

            def pair(p, _):
                start(2 * p + 1, 1)
                finish(0)

                @pl.when(p < nch // 2 - 1)
                def _():
                    start(2 * p + 2, 0)
                finish(1)
                return 0
            lax.fori_loop(0, nch // 2, pair, 0)
            plsc.subcore_barrier()

            def wcp(i, _):
                pltpu.sync_copy(acc_sh.at[pl.ds(zbase + i * 64, 64)], buf)

                def q8(q, _):
                    rc0 = rec0_v[pl.ds(i * 64 + q * 16, 16)]
                    rc1 = rec1_v[pl.ds(i * 64 + q * 16, 16)]
                    for l in range(16):
                        m0 = jnp.full((16,), rc0[l], jnp.float32)
                        m1 = jnp.full((16,), rc1[l], jnp.float32)
                        row = q * 16 + l
                        for v in range(8):
                            m = m0 if v < 4 else m1
                            buf[row, pl.ds(v * 16, 16)] = (
                                buf[row, pl.ds(v * 16, 16)] * m)
                    return 0
                lax.fori_loop(0, 4, q8, 0)
                pltpu.sync_copy(buf, num_hbm.at[pl.ds(zbase + i * 64, 64)])
                return 0
            lax.fori_loop(0, nzc, wcp, 0)
            plsc.subcore_barrier()

        for cc in range(NC):
            @pl.when(cid == cc)
            def _():
                for jj in range(2):
                    hp = cc * 2 + jj
                    job(hrefs[hp], erefs[hp], rrefs[hp], numrefs[hp])

    return pl.kernel(
        body,
        out_type=[jax.ShapeDtypeStruct((npad, 128), jnp.float32)] * 4,
        mesh=mesh,
        compiler_params=_SC_PARAMS,
        scratch_types=[
            pltpu.VMEM_SHARED((npad, 128), jnp.float32),
            pltpu.VMEM((npad // NT,), jnp.float32),
            pltpu.VMEM((npad // NT,), jnp.float32),
            pltpu.VMEM((2 * C,), jnp.float32),
            pltpu.VMEM((2 * C,), jnp.float32),
            pltpu.VMEM((C,), jnp.int32),
            pltpu.VMEM((C,), jnp.int32),
            pltpu.VMEM((C,), jnp.int32),
            pltpu.VMEM((C,), jnp.int32),
            pltpu.VMEM((C, 128), jnp.float32),
            pltpu.VMEM((C, 128), jnp.float32),
            pltpu.VMEM((64, 128), jnp.float32),
            pltpu.SemaphoreType.DMA,
            pltpu.SemaphoreType.DMA,
        ],
    )


def _make_sc_alpha2(npad, epad):
    epw = epad // (NT * NC)
    CA = 432
    GA = CA // 16
    nch = epw // CA
    mesh = plsc.VectorSubcoreMesh(core_axis_name="c", subcore_axis_name="s")

    def body(src_h, dst_h, ts_h, td_h, e_h, dena_h, denb_h,
             ts_v, td_v, denp_v, ebuf, sidx, didx, densh):
        cid = lax.axis_index("c")
        sid = lax.axis_index("s")
        wid = cid * NT + sid
        iota = lax.broadcasted_iota(jnp.int32, (16,), 0)
        seg = npad // NT
        pltpu.sync_copy(ts_h, ts_v)
        pltpu.sync_copy(td_h, td_v)
        _zero_1d(denp_v, npad)

        def chunk(k, _):
            base = (wid * nch + k) * CA
            pltpu.sync_copy(src_h.at[pl.ds(base, CA)], sidx)
            pltpu.sync_copy(dst_h.at[pl.ds(base, CA)], didx)

            def group(g, _):
                sv = sidx[pl.ds(g * 16, 16)]
                dv = didx[pl.ds(g * 16, 16)]
                a = plsc.load_gather(ts_v, [sv])
                b = plsc.load_gather(td_v, [dv])
                ev = _leaky_exp(a + b)
                ebuf[pl.ds(g * 16, 16)] = ev
                for l in range(16):
                    eb = jnp.full((16,), ev[l], jnp.float32)
                    di = jnp.full((16,), dv[l], jnp.int32)
                    plsc.addupdate_scatter(denp_v, [di], eb,
                                           mask=iota < 1)
                return 0
            lax.fori_loop(0, GA, group, 0)
            pltpu.sync_copy(ebuf, e_h.at[pl.ds(base, CA)])
            return 0
        lax.fori_loop(0, nch, chunk, 0)
        pltpu.sync_copy(denp_v, densh.at[pl.ds(sid * npad, npad)])
        plsc.subcore_barrier()

        off = sid * seg
        for r in range(NT):
            pltpu.sync_copy(densh.at[pl.ds(r * npad + off, seg)],
                            denp_v.at[pl.ds(r * seg, seg)])

        def red(j, _):
            acc = denp_v[pl.ds(j * 16, 16)]
            for r in range(1, NT):
                acc = acc + denp_v[pl.ds(r * seg + j * 16, 16)]
            denp_v[pl.ds(j * 16, 16)] = acc
            return 0
        lax.fori_loop(0, seg // 16, red, 0)
        for cc in range(NC):
            @pl.when(cid == cc)
            def _():
                pltpu.sync_copy(denp_v.at[pl.ds(0, seg)],
                                [dena_h, denb_h][cc].at[pl.ds(off, seg)])

    return pl.kernel(
        body,
        out_type=[jax.ShapeDtypeStruct((epad,), jnp.float32),
                  jax.ShapeDtypeStruct((npad,), jnp.float32),
                  jax.ShapeDtypeStruct((npad,), jnp.float32)],
        mesh=mesh,
        compiler_params=_SC_PARAMS,
        scratch_types=[
            pltpu.VMEM((npad,), jnp.float32),
            pltpu.VMEM((npad,), jnp.float32),
            pltpu.VMEM((npad,), jnp.float32),
            pltpu.VMEM((432,), jnp.float32),
            pltpu.VMEM((432,), jnp.int32),
            pltpu.VMEM((432,), jnp.int32),
            pltpu.VMEM_SHARED((NT * npad,), jnp.float32),
        ],
    )


def _tcr2_body(d_ref, r_ref):
    s = jnp.sum(d_ref[...], axis=0)
    r_ref[...] = (1.0 / (s + 1e-16)).reshape(1, BLK)


def _tcr2(den, npad):
    return pl.pallas_call(
        _tcr2_body,
        grid=(npad // BLK,),
        in_specs=[pl.BlockSpec((NT * NC, BLK), lambda i: (0, i))],
        out_specs=pl.BlockSpec((1, BLK), lambda i: (0, i)),
        out_shape=jax.ShapeDtypeStruct((1, npad), jnp.float32),
    )(den.reshape(NT * NC, npad))


def _make_sc_agg2(npad, epad):
    rpt = npad // NT
    nzc = rpt // 64
    nch = epad // C // (NT * NC)
    mesh = plsc.VectorSubcoreMesh(core_axis_name="c", subcore_axis_name="s")

    def body(src_h, dst_h, h2t, e_h, dena_h, denb_h, numa, numb,
             acc_sh, rec_v, tmp_v, ebuf_a, ebuf_b, sidx_a, sidx_b,
             didx_a, didx_b, rows_a, rows_b, buf,
             sem_a, sem_b):
        cid = lax.axis_index("c")
        sid = lax.axis_index("s")
        zbase = sid * rpt
        wid = cid * NT + sid
        sems = [sem_a, sem_b]
        ebufs = [ebuf_a, ebuf_b]
        sidxs = [sidx_a, sidx_b]
        didxs = [didx_a, didx_b]
        rowss = [rows_a, rows_b]
        pltpu.sync_copy(dena_h.at[pl.ds(sid * rpt, rpt)], rec_v)
        pltpu.sync_copy(denb_h.at[pl.ds(sid * rpt, rpt)], tmp_v)

        def mkrec(j, _):
            a = rec_v[pl.ds(j * 16, 16)]
            b = tmp_v[pl.ds(j * 16, 16)]
            rec_v[pl.ds(j * 16, 16)] = 1.0 / (a + b + 1e-16)
            return 0
        lax.fori_loop(0, rpt // 16, mkrec, 0)

        def zrow(i, _):
            for v in range(8):
                buf[i, pl.ds(v * 16, 16)] = jnp.zeros((16,), jnp.float32)
            return 0
        lax.fori_loop(0, 64, zrow, 0)

        def zcp(i, _):
            pltpu.sync_copy(buf, acc_sh.at[pl.ds(zbase + i * 64, 64)])
            return 0
        lax.fori_loop(0, nzc, zcp, 0)
        plsc.subcore_barrier()

        def start(k, b):
            base = (wid * nch + k) * C
            pltpu.sync_copy(src_h.at[pl.ds(base, C)], sidxs[b])
            pltpu.sync_copy(dst_h.at[pl.ds(base, C)], didxs[b])
            pltpu.sync_copy(e_h.at[pl.ds(base, C)], ebufs[b])
            pltpu.make_async_copy(h2t.at[sidxs[b]],
                                  rowss[b], sems[b]).start()

        def finish(b):
            pltpu.make_async_copy(h2t.at[sidxs[b]],
                                  rowss[b], sems[b]).wait()
            rows = rowss[b]
            ebuf = ebufs[b]

            def group(g, _):
                ev = ebuf[pl.ds(g * 16, 16)]
                for l in range(16):
                    e = jnp.full((16,), ev[l], jnp.float32)
                    r = g * 16 + l
                    for v in range(8):
                        rows[r, pl.ds(v * 16, 16)] = (
                            rows[r, pl.ds(v * 16, 16)] * e)
                return 0
            lax.fori_loop(0, G, group, 0)
            pltpu.sync_copy(rows, acc_sh.at[didxs[b]], add=True)

        start(0, 0)

        def pair(p, _):
            start(2 * p + 1, 1)
            finish(0)
            if nch % 2 == 1:
                start(2 * p + 2, 0)
            else:
                @pl.when(p < nch // 2 - 1)
                def _():
                    start(2 * p + 2, 0)
            finish(1)
            return 0
        lax.fori_loop(0, nch // 2, pair, 0)
        if nch % 2 == 1:
            finish(0)
        plsc.subcore_barrier()

        def wcp_core(num_hbm):
            def wcp(i, _):
                pltpu.sync_copy(acc_sh.at[pl.ds(zbase + i * 64, 64)], buf)

                def q16(q, _):
                    rc = rec_v[pl.ds(i * 64 + q * 16, 16)]
                    for l in range(16):
                        m = jnp.full((16,), rc[l], jnp.float32)
                        row = q * 16 + l
                        for v in range(8):
                            buf[row, pl.ds(v * 16, 16)] = (
                                buf[row, pl.ds(v * 16, 16)] * m)
                    return 0
                lax.fori_loop(0, 4, q16, 0)
                pltpu.sync_copy(buf, num_hbm.at[pl.ds(zbase + i * 64, 64)])
                return 0
            lax.fori_loop(0, nzc, wcp, 0)

        for cc in range(NC):
            @pl.when(cid == cc)
            def _():
                wcp_core([numa, numb][cc])

    return pl.kernel(
        body,
        out_type=[jax.ShapeDtypeStruct((npad, 128), jnp.float32)] * 2,
        mesh=mesh,
        compiler_params=_SC_PARAMS,
        scratch_types=[
            pltpu.VMEM_SHARED((npad, 128), jnp.float32),
            pltpu.VMEM((npad // NT,), jnp.float32),
            pltpu.VMEM((npad // NT,), jnp.float32),
            pltpu.VMEM((C,), jnp.float32),
            pltpu.VMEM((C,), jnp.float32),
            pltpu.VMEM((C,), jnp.int32),
            pltpu.VMEM((C,), jnp.int32),
            pltpu.VMEM((C,), jnp.int32),
            pltpu.VMEM((C,), jnp.int32),
            pltpu.VMEM((C, 128), jnp.float32),
            pltpu.VMEM((C, 128), jnp.float32),
            pltpu.VMEM((64, 128), jnp.float32),
            pltpu.SemaphoreType.DMA,
            pltpu.SemaphoreType.DMA,
        ],
    )


def _tc1_body(n0, n1, n2, n3, b1_ref, w2_ref, ms2_ref, md2_ref,
              h2_ref, tsf_ref, tdf_ref):
    h_mid = jnp.concatenate(
        [n0[...], n1[...], n2[...], n3[...]], axis=1) + b1_ref[...]
    h_mid = jnp.maximum(h_mid, 0.0)
    h2 = h_mid @ w2_ref[...]
    h2_ref[...] = h2
    tsf_ref[...] = h2 @ ms2_ref[...]
    tdf_ref[...] = h2 @ md2_ref[...]


def _tc1(nums, b1, W2, Ms2, Md2, npad):
    fb = pl.BlockSpec((BLK, 128), lambda i: (i, 0))
    return pl.pallas_call(
        _tc1_body,
        grid=(npad // BLK,),
        in_specs=[fb] * 4 + [
            pl.BlockSpec((1, 512), lambda i: (0, 0)),
            pl.BlockSpec((512, 128), lambda i: (0, 0)),
            pl.BlockSpec((128, 128), lambda i: (0, 0)),
            pl.BlockSpec((128, 128), lambda i: (0, 0)),
        ],
        out_specs=[fb, fb, fb],
        out_shape=[jax.ShapeDtypeStruct((npad, 128), jnp.float32)] * 3,
    )(*nums, b1.reshape(1, 512), W2, Ms2, Md2)


def _make_tc2_body(n_real, nblocks):
    def body(na, nb, b2_ref, p1_ref, pb1_ref, p2_ref, pb2_ref,
             zp_ref, gsum_ref, gp_ref):
        i = pl.program_id(0)
        z = na[...] + nb[...] + b2_ref[...]
        zp_ref[...] = jnp.maximum(z @ p1_ref[...] + pb1_ref[...],
                                  0.0) @ p2_ref[...] + pb2_ref[...]
        rid = BLK * i + lax.broadcasted_iota(jnp.int32, (BLK, 1), 0)
        part = jnp.sum(jnp.where(rid < n_real, z, 0.0), axis=0,
                       keepdims=True)

        @pl.when(i == 0)
        def _():
            gsum_ref[...] = part

        @pl.when(i > 0)
        def _():
            gsum_ref[...] = gsum_ref[...] + part

        @pl.when(i == nblocks - 1)
        def _():
            g = gsum_ref[...] / float(n_real)
            gp_ref[...] = jnp.maximum(g @ p1_ref[...] + pb1_ref[...],
                                      0.0) @ p2_ref[...] + pb2_ref[...]
    return body


def _tc2(numa, numb, b2, P1, pb1, P2, pb2, n_real, npad):
    nblocks = npad // BLK
    fb = pl.BlockSpec((BLK, 128), lambda i: (i, 0))
    one = pl.BlockSpec((1, 128), lambda i: (0, 0))
    return pl.pallas_call(
        _make_tc2_body(n_real, nblocks),
        grid=(nblocks,),
        in_specs=[fb, fb,
                  pl.BlockSpec((1, 128), lambda i: (0, 0)),
                  pl.BlockSpec((128, 64), lambda i: (0, 0)),
                  pl.BlockSpec((1, 64), lambda i: (0, 0)),
                  pl.BlockSpec((64, 128), lambda i: (0, 0)),
                  pl.BlockSpec((1, 128), lambda i: (0, 0))],
        out_specs=[fb, one, one],
        out_shape=[jax.ShapeDtypeStruct((npad, 128), jnp.float32),
                   jax.ShapeDtypeStruct((1, 128), jnp.float32),
                   jax.ShapeDtypeStruct((1, 128), jnp.float32)],
    )(numa, numb, b2.reshape(1, 128), P1, pb1.reshape(1, 64),
      P2, pb2.reshape(1, 128))


def _logit_mat(a, heads, ch):
    af = a.reshape(heads, ch)
    cols = []
    for c in range(128):
        h_ = (c // 16) % heads
        v = jnp.zeros((heads * ch,), jnp.float32)
        v = v.at[ch * h_:ch * (h_ + 1)].set(af[h_])
        cols.append(v)
    return jnp.stack(cols, axis=1)


def kernel(x, edge_index, W1, a_src1, a_dst1, b1, W2, a_src2, a_dst2, b2,
           P1, pb1, P2, pb2):
    N, _ = x.shape
    E = edge_index.shape[1]
    npad = ((N + (64 * NT) - 1) // (64 * NT)) * (64 * NT)
    gran = C * NT * NC
    epad = ((E + N + gran - 1) // gran) * gran

    xp = jnp.pad(x, ((0, npad - N), (0, 0)))
    sl = jnp.arange(N, dtype=jnp.int32)
    pad_n = epad - E - N
    src = jnp.concatenate(
        [edge_index[0].astype(jnp.int32), sl,
         jnp.zeros((pad_n,), jnp.int32)])
    dst = jnp.concatenate(
        [edge_index[1].astype(jnp.int32), sl,
         jnp.full((pad_n,), N, jnp.int32)])

    Ms = _logit_mat(a_src1, 8, 64)
    Md = _logit_mat(a_dst1, 8, 64)
    h1p0, h1p1, h1p2, h1p3, tsf, tdf = _tc0(xp, W1, Ms, Md, npad)

    tabs = []
    for j in range(4):
        t = jnp.stack([tsf[:, 32 * j], tsf[:, 32 * j + 16],
                       tdf[:, 32 * j], tdf[:, 32 * j + 16]], axis=1)
        tabs.append(t.reshape(-1))

    a1 = _make_sc_alpha1(npad, epad)(src, dst, *tabs)
    evals1, recs1 = a1[0:4], a1[4:8]

    sc_agg1 = _make_sc_agg1(npad, epad)
    nums = sc_agg1(src, dst, h1p0, h1p1, h1p2, h1p3, *evals1, *recs1)

    Ms2 = jnp.tile(a_src2.reshape(128, 1), (1, 128))
    Md2 = jnp.tile(a_dst2.reshape(128, 1), (1, 128))
    h2, tsf2, tdf2 = _tc1(nums, b1, W2, Ms2, Md2, npad)

    a2 = _make_sc_alpha2(npad, epad)(src, dst, tsf2[:, 0], tdf2[:, 0])
    evals2, dena2, denb2 = a2

    numa, numb = _make_sc_agg2(npad, epad)(src, dst, h2, evals2,
                                           dena2, denb2)

    zp, _, gp = _tc2(numa, numb, b2, P1, pb1, P2, pb2, N, npad)
    return (zp[:N], gp)
